# baseline (speedup 1.0000x reference)
import functools

import jax
import jax.numpy as jnp
from jax import lax
from jax.experimental import pallas as pl
from jax.experimental.pallas import tpu as pltpu
from jax.experimental.pallas import tpu_sc as plsc

NN = 10000
EE = 320000
NFEAT = 128
NHID = 64
NCLS = 40
ALPHA = 0.2
NC, NS, L = 2, 16, 16
CHUNK = 80
NG = CHUNK // L
RPT = 1000
NTD = NN // RPT
NB = 1000


def _elu(v):
    return jnp.where(v > 0, v, jnp.exp(jnp.minimum(v, 0.0)) - 1.0)


def _pre_body(x_ref, wc_ref, as_ref, ad_ref, hf_ref, fs_ref, fd_ref):
    h = jnp.dot(x_ref[...], wc_ref[...], preferred_element_type=jnp.float32)
    hf_ref[0] = h[:, :128]
    hf_ref[1] = h[:, 128:]
    fs_ref[...] = jnp.dot(h, as_ref[...], preferred_element_type=jnp.float32)
    fd_ref[...] = jnp.dot(h, ad_ref[...], preferred_element_type=jnp.float32)


def _pre(x, wcat, asrc, adst):
    return pl.pallas_call(
        _pre_body,
        grid=(NN // NB,),
        in_specs=[
            pl.BlockSpec((NB, NFEAT), lambda i: (i, 0)),
            pl.BlockSpec((NFEAT, 256), lambda i: (0, 0)),
            pl.BlockSpec((256, 16), lambda i: (0, 0)),
            pl.BlockSpec((256, 16), lambda i: (0, 0)),
        ],
        out_specs=[
            pl.BlockSpec((2, NB, 128), lambda i: (0, i, 0)),
            pl.BlockSpec((NB, 16), lambda i: (i, 0)),
            pl.BlockSpec((NB, 16), lambda i: (i, 0)),
        ],
        out_shape=[
            jax.ShapeDtypeStruct((2, NN, 128), jnp.float32),
            jax.ShapeDtypeStruct((NN, 16), jnp.float32),
            jax.ShapeDtypeStruct((NN, 16), jnp.float32),
        ],
    )(x, wcat, asrc, adst)


def _mid_body(hp_ref, rs_ref, wo_ref, as2_ref, ad2_ref, g_ref, gs_ref, gd_ref):
    parts = []
    for head in range(4):
        c, j = divmod(head, 2)
        hp = hp_ref[c][:, j * NHID:(j + 1) * NHID]
        denom = rs_ref[:, head][:, None] + 1e-9
        parts.append(_elu(hp / denom))
    x1 = jnp.concatenate(parts, axis=1)
    g = jnp.dot(x1, wo_ref[...], preferred_element_type=jnp.float32)
    g_ref[...] = g
    gs_ref[...] = jnp.dot(g, as2_ref[...], preferred_element_type=jnp.float32)
    gd_ref[...] = jnp.dot(g, ad2_ref[...], preferred_element_type=jnp.float32)


def _mid(hp, rs, wo128, as2, ad2):
    return pl.pallas_call(
        _mid_body,
        grid=(NN // NB,),
        in_specs=[
            pl.BlockSpec((2, NB, 128), lambda i: (0, i, 0)),
            pl.BlockSpec((NB, 4), lambda i: (i, 0)),
            pl.BlockSpec((256, 128), lambda i: (0, 0)),
            pl.BlockSpec((128, 16), lambda i: (0, 0)),
            pl.BlockSpec((128, 16), lambda i: (0, 0)),
        ],
        out_specs=[
            pl.BlockSpec((NB, 128), lambda i: (i, 0)),
            pl.BlockSpec((NB, 16), lambda i: (i, 0)),
            pl.BlockSpec((NB, 16), lambda i: (i, 0)),
        ],
        out_shape=[
            jax.ShapeDtypeStruct((NN, 128), jnp.float32),
            jax.ShapeDtypeStruct((NN, 16), jnp.float32),
            jax.ShapeDtypeStruct((NN, 16), jnp.float32),
        ],
    )(hp, rs, wo128, as2, ad2)


def _post_body(acc_ref, rs_ref, o_ref):
    comb = acc_ref[0][:, :NCLS] + acc_ref[1][:, :NCLS]
    rsum = rs_ref[0] + rs_ref[1] + 1e-9
    o = _elu(comb / rsum)
    m = jnp.max(o, axis=1, keepdims=True)
    lse = jnp.log(jnp.sum(jnp.exp(o - m), axis=1, keepdims=True))
    o_ref[...] = o - m - lse


def _post(acc2, rs2):
    return pl.pallas_call(
        _post_body,
        grid=(NN // NB,),
        in_specs=[
            pl.BlockSpec((2, NB, 128), lambda i: (0, i, 0)),
            pl.BlockSpec((2, NB, 1), lambda i: (0, i, 0)),
        ],
        out_specs=pl.BlockSpec((NB, NCLS), lambda i: (i, 0)),
        out_shape=jax.ShapeDtypeStruct((NN, NCLS), jnp.float32),
    )(acc2, rs2)


def _make_edge_pass(pair):
    ept = EE // NS if pair else EE // (NC * NS)
    nchunk = ept // CHUNK
    nj = 2 if pair else 1
    nscale = 8 if pair else 3
    shift = 6 if pair else 7
    nmask = 63 if pair else 127
    lmul = 2 if pair else 1
    nrs = 160 if pair else 80

    mesh = plsc.VectorSubcoreMesh(
        core_axis_name="c", subcore_axis_name="s",
        num_cores=NC, num_subcores=NS)

    @functools.partial(
        pl.kernel,
        out_type=[
            jax.ShapeDtypeStruct((NC, NN, 128), jnp.float32),
            jax.ShapeDtypeStruct((NC, nrs, 128), jnp.float32),
        ],
        mesh=mesh,
        compiler_params=pltpu.CompilerParams(needs_layout_passes=False),
        scratch_types=[
            pltpu.VMEM((CHUNK,), jnp.int32),
            pltpu.VMEM((CHUNK,), jnp.int32),
            pltpu.VMEM((CHUNK,), jnp.int32),
            pltpu.VMEM((CHUNK,), jnp.int32),
            pltpu.VMEM((CHUNK,), jnp.int32),
            pltpu.VMEM((CHUNK,), jnp.int32),
            pltpu.VMEM((CHUNK,), jnp.int32),
            pltpu.VMEM((CHUNK,), jnp.int32),
            pltpu.VMEM((NN,), jnp.int32 if pair else jnp.float32),
            pltpu.VMEM((NN,), jnp.int32 if pair else jnp.float32),
            pltpu.VMEM((CHUNK, 128), jnp.float32),
            pltpu.VMEM((CHUNK, 128), jnp.float32),
            pltpu.VMEM((CHUNK * L,), jnp.float32),
            pltpu.VMEM_SHARED((NN, 128), jnp.float32),
            pltpu.VMEM_SHARED((nrs, 128), jnp.float32),
            pltpu.SemaphoreType.DMA,
            pltpu.SemaphoreType.DMA,
            pltpu.SemaphoreType.DMA,
            pltpu.SemaphoreType.DMA,
        ],
    )
    def edge_pass(adj, logits_hbm, tab_hbm,
                  acc_out, rs_out,
                  src_v0, src_v1, dst_v0, dst_v1, gidx_v0, gidx_v1,
                  srow_v0, srow_v1, fsT, fdT,
                  t_b, rs_b, e_bf, acc, rsacc, gsem, asem, rsem, isem):
        srcs = (src_v0, src_v1)
        dsts = (dst_v0, dst_v1)
        gidxs = (gidx_v0, gidx_v1)
        srows = (srow_v0, srow_v1)
        c = lax.axis_index("c")
        s = lax.axis_index("s")

        if pair:
            pltpu.sync_copy(logits_hbm.at[pl.ds(c * NN, NN)], fsT)
            pltpu.sync_copy(logits_hbm.at[pl.ds((2 + c) * NN, NN)], fdT)
        else:
            pltpu.sync_copy(logits_hbm.at[pl.ds(0, NN)], fsT)
            pltpu.sync_copy(logits_hbm.at[pl.ds(NN, NN)], fdT)

        def z_body(i, cr):
            for j in range(8):
                rs_b[i, pl.ds(j * L, L)] = jnp.zeros((L,), jnp.float32)
            return cr
        lax.fori_loop(0, CHUNK, z_body, 0)

        def zacc_body(i, cr):
            m = s + 16 * i

            @pl.when(m < NN // CHUNK)
            def _():
                pltpu.sync_copy(rs_b, acc.at[pl.ds(m * CHUNK, CHUNK)])
            return cr
        lax.fori_loop(0, (NN // CHUNK + 15) // 16, zacc_body, 0)

        @pl.when(s < nrs // CHUNK)
        def _init_rs():
            pltpu.sync_copy(rs_b, rsacc.at[pl.ds(s * CHUNK, CHUNK)])
        plsc.subcore_barrier()

        lane = lax.iota(jnp.int32, L)
        tile_e0 = s * ept if pair else c * (EE // NC) + s * ept
        zeros16 = jnp.zeros((L,), jnp.float32)
        zidx = jnp.zeros((L,), jnp.int32)
        bdnums = lax.GatherDimensionNumbers(
            offset_dims=(), collapsed_slice_dims=(0,), start_index_map=(0,))

        def bcast(v, iv):
            return lax.gather(v, iv[:, None], bdnums, (1,),
                              mode=lax.GatherScatterMode.PROMISE_IN_BOUNDS)

        def issue_idx(k, st):
            base = tile_e0 + k * CHUNK
            return (pltpu.async_copy(adj.at[pl.ds(base, CHUNK)],
                                     srcs[st], isem),
                    pltpu.async_copy(adj.at[pl.ds(EE + base, CHUNK)],
                                     dsts[st], isem))

        def prep_idx(st):
            for j in range(NG):
                sl = pl.ds(j * L, L)
                srows[st][sl] = lax.shift_right_logical(srcs[st][sl], shift)
            if pair:
                for j in range(NG):
                    sl = pl.ds(j * L, L)
                    gidxs[st][sl] = dsts[st][sl] + c * NN

        def chunk_step(k, cur):
            nxt = 1 - cur
            @pl.when(k > 0)
            def _():
                pltpu.make_async_copy(t_b, acc.at[srcs[nxt]], asem).wait()
            gcp = pltpu.async_copy(
                tab_hbm.at[gidxs[cur] if pair else dsts[cur]], t_b, gsem)

            @pl.when(k + 1 < nchunk)
            def _():
                issue_idx(k + 1, nxt)

            def e_body(g, cr):
                sidx = srcs[cur][pl.ds(g * L, L)]
                didx = dsts[cur][pl.ds(g * L, L)]
                rowi = g * L + lane
                cbase = (sidx & nmask) * lmul
                ebase = rowi * L
                ws = plsc.load_gather(fsT, [sidx])
                wd = plsc.load_gather(fdT, [didx])
                if pair:
                    hi = jnp.int32(-65536)
                    zs = (plsc.bitcast(jnp.left_shift(ws, 16), jnp.float32),
                          plsc.bitcast(jnp.bitwise_and(ws, hi), jnp.float32))
                    zd = (plsc.bitcast(jnp.left_shift(wd, 16), jnp.float32),
                          plsc.bitcast(jnp.bitwise_and(wd, hi), jnp.float32))
                else:
                    zs = (ws,)
                    zd = (wd,)
                for j in range(nj):
                    z = zs[j] + zd[j]
                    ev = jnp.exp(-jnp.maximum(z, z * ALPHA))
                    plsc.store_scatter(rs_b, [rowi, cbase + j], ev)
                    plsc.store_scatter(e_bf, [ebase + j], ev)
                return cr
            for g_ in range(NG):
                e_body(g_, 0)

            rcp = pltpu.async_copy(rs_b, rsacc.at[srows[cur]], rsem, add=True)

            @pl.when(k + 1 < nchunk)
            def _():
                base = tile_e0 + (k + 1) * CHUNK
                pltpu.make_async_copy(adj.at[pl.ds(base, CHUNK)],
                                      srcs[nxt], isem).wait()
                pltpu.make_async_copy(adj.at[pl.ds(EE + base, CHUNK)],
                                      dsts[nxt], isem).wait()
                prep_idx(nxt)

            gcp.wait()

            def s_body(i, cr):
                ev = e_bf[pl.ds(i * L, L)]
                eA = bcast(ev, zidx)
                eB = bcast(ev, zidx + 1) if pair else eA
                for j in range(nscale):
                    sl = pl.ds(j * L, L)
                    ee = eA if (not pair or j < 4) else eB
                    t_b[i, sl] = t_b[i, sl] * ee
                return cr
            lax.fori_loop(0, CHUNK, s_body, 0, unroll=8)

            pltpu.async_copy(t_b, acc.at[srcs[cur]], asem, add=True)

            rcp.wait()

            def uz_body(g, cr):
                sidx = srcs[cur][pl.ds(g * L, L)]
                rowi = g * L + lane
                cbase = (sidx & nmask) * lmul
                for j in range(nj):
                    plsc.store_scatter(rs_b, [rowi, cbase + j], zeros16)
                return cr
            for g_ in range(NG):
                uz_body(g_, 0)

        ia, ib = issue_idx(jnp.int32(0), 0)
        ia.wait()
        ib.wait()
        prep_idx(0)

        def pair_body(i, carry):
            chunk_step(2 * i, 0)
            chunk_step(2 * i + 1, 1)
            return carry
        lax.fori_loop(0, nchunk // 2, pair_body, 0)
        if nchunk % 2:
            chunk_step(jnp.int32(nchunk - 1), 0)
            last = 0
        else:
            last = 1
        pltpu.make_async_copy(t_b, acc.at[srcs[last]], asem).wait()

        plsc.subcore_barrier()

        @pl.when(s < NTD)
        def _drain_acc():
            sl = pl.ds(s * RPT, RPT)
            pltpu.sync_copy(acc.at[sl], acc_out.at[c, sl])

        @pl.when(s == 0)
        def _drain_rs():
            pltpu.sync_copy(rsacc, rs_out.at[c])

    return edge_pass


_edge_pass1 = _make_edge_pass(True)
_edge_pass2 = _make_edge_pass(False)


def kernel(x, adj, W0, W1, W2, W3, a0, a1, a2, a3, Wout, aout):
    f32 = jnp.float32
    wcat = jnp.concatenate([W0, W1, W2, W3], axis=1)
    asrc = jnp.zeros((256, 16), f32)
    adst = jnp.zeros((256, 16), f32)
    for h, a in enumerate([a0, a1, a2, a3]):
        asrc = asrc.at[h * NHID:(h + 1) * NHID, h].set(a[:NHID])
        adst = adst.at[h * NHID:(h + 1) * NHID, h].set(a[NHID:])
    wo128 = jnp.zeros((256, 128), f32).at[:, :NCLS].set(Wout)
    as2 = jnp.zeros((128, 16), f32).at[:NCLS, 0].set(aout[:NCLS])
    ad2 = jnp.zeros((128, 16), f32).at[:NCLS, 0].set(aout[NCLS:])

    hflat, fs16, fd16 = _pre(x, wcat, asrc, adst)

    def pack2(a, b):
        ab = lax.bitcast_convert_type(a.astype(jnp.bfloat16), jnp.uint16)
        bb = lax.bitcast_convert_type(b.astype(jnp.bfloat16), jnp.uint16)
        w = ab.astype(jnp.uint32) | (bb.astype(jnp.uint32) << 16)
        return lax.bitcast_convert_type(w, jnp.int32)

    logits1 = jnp.concatenate(
        [pack2(fs16[:, 0], fs16[:, 1]), pack2(fs16[:, 2], fs16[:, 3]),
         pack2(fd16[:, 0], fd16[:, 1]), pack2(fd16[:, 2], fd16[:, 3])])
    adjf = adj.reshape(2 * EE)
    hp, rs1 = _edge_pass1(adjf, logits1, hflat.reshape(2 * NN, 128))
    rs4 = rs1.reshape(NC, 160 * 64, 2)[:, :NN, :].transpose(1, 0, 2).reshape(NN, 4)
    g128, gs16, gd16 = _mid(hp, rs4, wo128, as2, ad2)
    logits2 = jnp.concatenate([gs16[:, 0], gd16[:, 0]])
    acc2, rs2 = _edge_pass2(adjf, logits2, g128)
    rs2u = rs2.reshape(NC, 80 * 128)[:, :NN].reshape(NC, NN, 1)
    out = _post(acc2, rs2u)
    return out


# speedup vs baseline: 1.2121x; 1.2121x over previous
import functools

import jax
import jax.numpy as jnp
from jax import lax
from jax.experimental import pallas as pl
from jax.experimental.pallas import tpu as pltpu
from jax.experimental.pallas import tpu_sc as plsc

NN = 10000
EE = 320000
NFEAT = 128
NHID = 64
NCLS = 40
ALPHA = 0.2
NC, NS, L = 2, 16, 16
CHUNK = 64
NG = CHUNK // L
RPT = 1000
NTD = NN // RPT
NB = 1000


def _elu(v):
    return jnp.where(v > 0, v, jnp.exp(jnp.minimum(v, 0.0)) - 1.0)


def _pre_body(x_ref, wc_ref, as_ref, ad_ref, hf_ref, fs_ref, fd_ref):
    h = jnp.dot(x_ref[...], wc_ref[...], preferred_element_type=jnp.float32)
    hf_ref[0] = h[:, :128]
    hf_ref[1] = h[:, 128:]
    fs_ref[...] = jnp.dot(h, as_ref[...], preferred_element_type=jnp.float32)
    fd_ref[...] = jnp.dot(h, ad_ref[...], preferred_element_type=jnp.float32)


def _pre(x, wcat, asrc, adst):
    return pl.pallas_call(
        _pre_body,
        grid=(NN // NB,),
        in_specs=[
            pl.BlockSpec((NB, NFEAT), lambda i: (i, 0)),
            pl.BlockSpec((NFEAT, 256), lambda i: (0, 0)),
            pl.BlockSpec((256, 16), lambda i: (0, 0)),
            pl.BlockSpec((256, 16), lambda i: (0, 0)),
        ],
        out_specs=[
            pl.BlockSpec((2, NB, 128), lambda i: (0, i, 0)),
            pl.BlockSpec((NB, 16), lambda i: (i, 0)),
            pl.BlockSpec((NB, 16), lambda i: (i, 0)),
        ],
        out_shape=[
            jax.ShapeDtypeStruct((2, NN, 128), jnp.float32),
            jax.ShapeDtypeStruct((NN, 16), jnp.float32),
            jax.ShapeDtypeStruct((NN, 16), jnp.float32),
        ],
    )(x, wcat, asrc, adst)


def _mid_body(hp_ref, rs_ref, wo_ref, as2_ref, ad2_ref, g_ref, gs_ref, gd_ref):
    parts = []
    for head in range(4):
        c, j = divmod(head, 2)
        hp = hp_ref[c][:, j * NHID:(j + 1) * NHID]
        denom = rs_ref[:, head][:, None] + 1e-9
        parts.append(_elu(hp / denom))
    x1 = jnp.concatenate(parts, axis=1)
    g = jnp.dot(x1, wo_ref[...], preferred_element_type=jnp.float32)
    g_ref[...] = g
    gs_ref[...] = jnp.dot(g, as2_ref[...], preferred_element_type=jnp.float32)
    gd_ref[...] = jnp.dot(g, ad2_ref[...], preferred_element_type=jnp.float32)


def _mid(hp, rs, wo128, as2, ad2):
    return pl.pallas_call(
        _mid_body,
        grid=(NN // NB,),
        in_specs=[
            pl.BlockSpec((2, NB, 128), lambda i: (0, i, 0)),
            pl.BlockSpec((NB, 4), lambda i: (i, 0)),
            pl.BlockSpec((256, 128), lambda i: (0, 0)),
            pl.BlockSpec((128, 16), lambda i: (0, 0)),
            pl.BlockSpec((128, 16), lambda i: (0, 0)),
        ],
        out_specs=[
            pl.BlockSpec((NB, 128), lambda i: (i, 0)),
            pl.BlockSpec((NB, 16), lambda i: (i, 0)),
            pl.BlockSpec((NB, 16), lambda i: (i, 0)),
        ],
        out_shape=[
            jax.ShapeDtypeStruct((NN, 128), jnp.float32),
            jax.ShapeDtypeStruct((NN, 16), jnp.float32),
            jax.ShapeDtypeStruct((NN, 16), jnp.float32),
        ],
    )(hp, rs, wo128, as2, ad2)


def _post_body(acc_ref, rs_ref, o_ref):
    comb = acc_ref[0][:, :NCLS] + acc_ref[1][:, :NCLS]
    rsum = rs_ref[0] + rs_ref[1] + 1e-9
    o = _elu(comb / rsum)
    m = jnp.max(o, axis=1, keepdims=True)
    lse = jnp.log(jnp.sum(jnp.exp(o - m), axis=1, keepdims=True))
    o_ref[...] = o - m - lse


def _post(acc2, rs2):
    return pl.pallas_call(
        _post_body,
        grid=(NN // NB,),
        in_specs=[
            pl.BlockSpec((2, NB, 128), lambda i: (0, i, 0)),
            pl.BlockSpec((2, NB, 1), lambda i: (0, i, 0)),
        ],
        out_specs=pl.BlockSpec((NB, NCLS), lambda i: (i, 0)),
        out_shape=jax.ShapeDtypeStruct((NN, NCLS), jnp.float32),
    )(acc2, rs2)


def _make_edge_pass(pair):
    ept = EE // NS if pair else EE // (NC * NS)
    nchunk = -(-ept // CHUNK)
    nj = 2 if pair else 1
    nscale = 8 if pair else 3
    shift = 6 if pair else 7
    nmask = 63 if pair else 127
    lmul = 2 if pair else 1
    nrs = 160 if pair else 80

    mesh = plsc.VectorSubcoreMesh(
        core_axis_name="c", subcore_axis_name="s",
        num_cores=NC, num_subcores=NS)

    @functools.partial(
        pl.kernel,
        out_type=[
            jax.ShapeDtypeStruct((NC, NN, 128), jnp.float32),
            jax.ShapeDtypeStruct((NC, nrs, 128), jnp.float32),
        ],
        mesh=mesh,
        compiler_params=pltpu.CompilerParams(needs_layout_passes=False),
        scratch_types=[
            pltpu.VMEM((CHUNK,), jnp.int32),
            pltpu.VMEM((CHUNK,), jnp.int32),
            pltpu.VMEM((CHUNK,), jnp.int32),
            pltpu.VMEM((CHUNK,), jnp.int32),
            pltpu.VMEM((CHUNK,), jnp.int32),
            pltpu.VMEM((CHUNK,), jnp.int32),
            pltpu.VMEM((NN,), jnp.int32 if pair else jnp.float32),
            pltpu.VMEM((NN,), jnp.int32 if pair else jnp.float32),
            pltpu.VMEM((CHUNK, 128), jnp.float32),
            pltpu.VMEM((CHUNK, 128), jnp.float32),
            pltpu.VMEM((CHUNK, 128), jnp.float32),
            pltpu.VMEM((CHUNK * 2 + L,), jnp.float32),
            pltpu.VMEM_SHARED((NN, 128), jnp.float32),
            pltpu.VMEM_SHARED((nrs, 128), jnp.float32),
            pltpu.SemaphoreType.DMA,
            pltpu.SemaphoreType.DMA,
            pltpu.SemaphoreType.DMA,
            pltpu.SemaphoreType.DMA,
            pltpu.SemaphoreType.DMA,
        ],
    )
    def edge_pass(adj, logits_hbm, tab_hbm,
                  acc_out, rs_out,
                  src_v0, src_v1, dst_v0, dst_v1,
                  srow_v0, srow_v1, fsT, fdT,
                  t_b0, t_b1, rs_b, e_bf, acc, rsacc,
                  gsem0, gsem1, asem, rsem, isem):
        srcs = (src_v0, src_v1)
        dsts = (dst_v0, dst_v1)
        srows = (srow_v0, srow_v1)
        t_bs = (t_b0, t_b1)
        gsems = (gsem0, gsem1)
        c = lax.axis_index("c")
        s = lax.axis_index("s")

        if pair:
            pltpu.sync_copy(logits_hbm.at[pl.ds(c * NN, NN)], fsT)
            pltpu.sync_copy(logits_hbm.at[pl.ds((2 + c) * NN, NN)], fdT)
        else:
            pltpu.sync_copy(logits_hbm.at[pl.ds(0, NN)], fsT)
            pltpu.sync_copy(logits_hbm.at[pl.ds(NN, NN)], fdT)

        def z_body(i, cr):
            for j in range(8):
                rs_b[i, pl.ds(j * L, L)] = jnp.zeros((L,), jnp.float32)
            return cr
        lax.fori_loop(0, CHUNK, z_body, 0)

        def zacc_body(i, cr):
            m = s + 16 * i

            @pl.when(m < NN // CHUNK)
            def _():
                pltpu.sync_copy(rs_b, acc.at[pl.ds(m * CHUNK, CHUNK)])
            return cr
        lax.fori_loop(0, (NN // CHUNK + 15) // 16, zacc_body, 0)
        if NN % CHUNK:
            @pl.when(s == NS - 1)
            def _init_acc_tail():
                pltpu.sync_copy(rs_b.at[pl.ds(0, NN % CHUNK)],
                                acc.at[pl.ds(NN - NN % CHUNK, NN % CHUNK)])

        @pl.when(s < nrs // 16)
        def _init_rs():
            pltpu.sync_copy(rs_b.at[pl.ds(0, 16)],
                            rsacc.at[pl.ds(s * 16, 16)])
        plsc.subcore_barrier()

        lane = lax.iota(jnp.int32, L)
        tile_e0 = s * ept if pair else c * (EE // NC) + s * ept
        zeros16 = jnp.zeros((L,), jnp.float32)
        zidx = jnp.zeros((L,), jnp.int32)
        bdnums = lax.GatherDimensionNumbers(
            offset_dims=(), collapsed_slice_dims=(0,), start_index_map=(0,))

        def bcast(v, iv):
            return lax.gather(v, iv[:, None], bdnums, (1,),
                              mode=lax.GatherScatterMode.PROMISE_IN_BOUNDS)

        def issue_idx(k, st):
            base = tile_e0 + k * CHUNK
            return (pltpu.async_copy(adj.at[pl.ds(base, CHUNK)],
                                     srcs[st], isem),
                    pltpu.async_copy(adj.at[pl.ds(EE + base, CHUNK)],
                                     dsts[st], isem))

        def prep_idx(st):
            for j in range(NG):
                sl = pl.ds(j * L, L)
                srows[st][sl] = lax.shift_right_logical(srcs[st][sl], shift)
            if pair:
                for j in range(NG):
                    sl = pl.ds(j * L, L)
                    dsts[st][sl] = dsts[st][sl] + c * NN

        def chunk_step(k, cur):
            nxt = 1 - cur
            t_b = t_bs[cur]
            @pl.when(k > 0)
            def _():
                pltpu.make_async_copy(t_bs[nxt], acc.at[srcs[nxt]],
                                      asem).wait()

            @pl.when(k + 1 < nchunk)
            def _():
                issue_idx(k + 1, nxt)

            def e_body(g, cr):
                sidx = srcs[cur][pl.ds(g * L, L)]
                didx = dsts[cur][pl.ds(g * L, L)]
                if pair:
                    didx = didx - c * NN
                rowi = g * L + lane
                cbase = (sidx & nmask) * lmul
                ebase = rowi * 2
                ws = plsc.load_gather(fsT, [sidx])
                wd = plsc.load_gather(fdT, [didx])
                if pair:
                    hi = jnp.int32(-65536)
                    zs = (plsc.bitcast(jnp.left_shift(ws, 16), jnp.float32),
                          plsc.bitcast(jnp.bitwise_and(ws, hi), jnp.float32))
                    zd = (plsc.bitcast(jnp.left_shift(wd, 16), jnp.float32),
                          plsc.bitcast(jnp.bitwise_and(wd, hi), jnp.float32))
                else:
                    zs = (ws,)
                    zd = (wd,)
                inb = (k * CHUNK + rowi) < ept
                for j in range(nj):
                    z = zs[j] + zd[j]
                    ev = jnp.where(inb, jnp.exp(-jnp.maximum(z, z * ALPHA)),
                                   0.0)
                    plsc.store_scatter(rs_b, [rowi, cbase + j], ev)
                    plsc.store_scatter(e_bf, [ebase + j], ev)
                return cr
            for g_ in range(NG):
                e_body(g_, 0)

            rcp = pltpu.async_copy(rs_b, rsacc.at[srows[cur]], rsem, add=True)

            @pl.when(k + 1 < nchunk)
            def _():
                base = tile_e0 + (k + 1) * CHUNK
                pltpu.make_async_copy(adj.at[pl.ds(base, CHUNK)],
                                      srcs[nxt], isem).wait()
                pltpu.make_async_copy(adj.at[pl.ds(EE + base, CHUNK)],
                                      dsts[nxt], isem).wait()
                prep_idx(nxt)
                pltpu.async_copy(tab_hbm.at[dsts[nxt]], t_bs[nxt],
                                 gsems[nxt])

            pltpu.make_async_copy(tab_hbm.at[dsts[cur]], t_b,
                                  gsems[cur]).wait()

            def s_body(i, cr):
                ev = e_bf[pl.ds(i * 2, L)]
                eA = bcast(ev, zidx)
                eB = bcast(ev, zidx + 1) if pair else eA
                for j in range(nscale):
                    sl = pl.ds(j * L, L)
                    ee = eA if (not pair or j < 4) else eB
                    t_b[i, sl] = t_b[i, sl] * ee
                return cr
            lax.fori_loop(0, CHUNK, s_body, 0, unroll=8)

            pltpu.async_copy(t_b, acc.at[srcs[cur]], asem, add=True)

            rcp.wait()

            def uz_body(g, cr):
                sidx = srcs[cur][pl.ds(g * L, L)]
                rowi = g * L + lane
                cbase = (sidx & nmask) * lmul
                for j in range(nj):
                    plsc.store_scatter(rs_b, [rowi, cbase + j], zeros16)
                return cr
            for g_ in range(NG):
                uz_body(g_, 0)

        ia, ib = issue_idx(jnp.int32(0), 0)
        ia.wait()
        ib.wait()
        prep_idx(0)
        pltpu.async_copy(tab_hbm.at[dsts[0]], t_bs[0], gsems[0])

        def pair_body(i, carry):
            chunk_step(2 * i, 0)
            chunk_step(2 * i + 1, 1)
            return carry
        lax.fori_loop(0, nchunk // 2, pair_body, 0)
        if nchunk % 2:
            chunk_step(jnp.int32(nchunk - 1), 0)
            last = 0
        else:
            last = 1
        pltpu.make_async_copy(t_bs[last], acc.at[srcs[last]], asem).wait()

        plsc.subcore_barrier()

        @pl.when(s < NTD)
        def _drain_acc():
            sl = pl.ds(s * RPT, RPT)
            pltpu.sync_copy(acc.at[sl], acc_out.at[c, sl])

        @pl.when(s == 0)
        def _drain_rs():
            pltpu.sync_copy(rsacc, rs_out.at[c])

    return edge_pass


_edge_pass1 = _make_edge_pass(True)
_edge_pass2 = _make_edge_pass(False)


def kernel(x, adj, W0, W1, W2, W3, a0, a1, a2, a3, Wout, aout):
    f32 = jnp.float32
    wcat = jnp.concatenate([W0, W1, W2, W3], axis=1)
    asrc = jnp.zeros((256, 16), f32)
    adst = jnp.zeros((256, 16), f32)
    for h, a in enumerate([a0, a1, a2, a3]):
        asrc = asrc.at[h * NHID:(h + 1) * NHID, h].set(a[:NHID])
        adst = adst.at[h * NHID:(h + 1) * NHID, h].set(a[NHID:])
    wo128 = jnp.zeros((256, 128), f32).at[:, :NCLS].set(Wout)
    as2 = jnp.zeros((128, 16), f32).at[:NCLS, 0].set(aout[:NCLS])
    ad2 = jnp.zeros((128, 16), f32).at[:NCLS, 0].set(aout[NCLS:])

    hflat, fs16, fd16 = _pre(x, wcat, asrc, adst)

    def pack2(a, b):
        ab = lax.bitcast_convert_type(a.astype(jnp.bfloat16), jnp.uint16)
        bb = lax.bitcast_convert_type(b.astype(jnp.bfloat16), jnp.uint16)
        w = ab.astype(jnp.uint32) | (bb.astype(jnp.uint32) << 16)
        return lax.bitcast_convert_type(w, jnp.int32)

    logits1 = jnp.concatenate(
        [pack2(fs16[:, 0], fs16[:, 1]), pack2(fs16[:, 2], fs16[:, 3]),
         pack2(fd16[:, 0], fd16[:, 1]), pack2(fd16[:, 2], fd16[:, 3])])
    adjf = jnp.concatenate(
        [adj.reshape(2 * EE), jnp.zeros((CHUNK,), jnp.int32)])
    hp, rs1 = _edge_pass1(adjf, logits1, hflat.reshape(2 * NN, 128))
    rs4 = rs1.reshape(NC, 160 * 64, 2)[:, :NN, :].transpose(1, 0, 2).reshape(NN, 4)
    g128, gs16, gd16 = _mid(hp, rs4, wo128, as2, ad2)
    logits2 = jnp.concatenate([gs16[:, 0], gd16[:, 0]])
    acc2, rs2 = _edge_pass2(adjf, logits2, g128)
    rs2u = rs2.reshape(NC, 80 * 128)[:, :NN].reshape(NC, NN, 1)
    out = _post(acc2, rs2u)
    return out


# speedup vs baseline: 1.3536x; 1.1168x over previous
import functools

import jax
import jax.numpy as jnp
from jax import lax
from jax.experimental import pallas as pl
from jax.experimental.pallas import tpu as pltpu
from jax.experimental.pallas import tpu_sc as plsc

NN = 10000
EE = 320000
NFEAT = 128
NHID = 64
NCLS = 40
ALPHA = 0.2
NC, NS, L = 2, 16, 16
CHUNK = 64
NG = CHUNK // L
RPT = 1000
NTD = NN // RPT
NB = 1000


def _elu(v):
    return jnp.where(v > 0, v, jnp.exp(jnp.minimum(v, 0.0)) - 1.0)


def _pre_body(x_ref, wc_ref, as_ref, ad_ref, hf_ref, fs_ref, fd_ref):
    h = jnp.dot(x_ref[...], wc_ref[...], preferred_element_type=jnp.float32)
    hf_ref[0] = h[:, :128]
    hf_ref[1] = h[:, 128:]
    fs_ref[...] = jnp.dot(h, as_ref[...], preferred_element_type=jnp.float32)
    fd_ref[...] = jnp.dot(h, ad_ref[...], preferred_element_type=jnp.float32)


def _pre(x, wcat, asrc, adst):
    return pl.pallas_call(
        _pre_body,
        grid=(NN // NB,),
        in_specs=[
            pl.BlockSpec((NB, NFEAT), lambda i: (i, 0)),
            pl.BlockSpec((NFEAT, 256), lambda i: (0, 0)),
            pl.BlockSpec((256, 16), lambda i: (0, 0)),
            pl.BlockSpec((256, 16), lambda i: (0, 0)),
        ],
        out_specs=[
            pl.BlockSpec((2, NB, 128), lambda i: (0, i, 0)),
            pl.BlockSpec((NB, 16), lambda i: (i, 0)),
            pl.BlockSpec((NB, 16), lambda i: (i, 0)),
        ],
        out_shape=[
            jax.ShapeDtypeStruct((2, NN, 128), jnp.float32),
            jax.ShapeDtypeStruct((NN, 16), jnp.float32),
            jax.ShapeDtypeStruct((NN, 16), jnp.float32),
        ],
    )(x, wcat, asrc, adst)


def _mid_body(hp_ref, rs_ref, wo_ref, as2_ref, ad2_ref, g_ref, gs_ref, gd_ref):
    parts = []
    for head in range(4):
        c, j = divmod(head, 2)
        hp = hp_ref[c][:, j * NHID:(j + 1) * NHID]
        denom = rs_ref[:, head][:, None] + 1e-9
        parts.append(_elu(hp / denom))
    x1 = jnp.concatenate(parts, axis=1)
    g = jnp.dot(x1, wo_ref[...], preferred_element_type=jnp.float32)
    g_ref[...] = g
    gs_ref[...] = jnp.dot(g, as2_ref[...], preferred_element_type=jnp.float32)
    gd_ref[...] = jnp.dot(g, ad2_ref[...], preferred_element_type=jnp.float32)


def _mid(hp, rs, wo128, as2, ad2):
    return pl.pallas_call(
        _mid_body,
        grid=(NN // NB,),
        in_specs=[
            pl.BlockSpec((2, NB, 128), lambda i: (0, i, 0)),
            pl.BlockSpec((NB, 4), lambda i: (i, 0)),
            pl.BlockSpec((256, 128), lambda i: (0, 0)),
            pl.BlockSpec((128, 16), lambda i: (0, 0)),
            pl.BlockSpec((128, 16), lambda i: (0, 0)),
        ],
        out_specs=[
            pl.BlockSpec((NB, 128), lambda i: (i, 0)),
            pl.BlockSpec((NB, 16), lambda i: (i, 0)),
            pl.BlockSpec((NB, 16), lambda i: (i, 0)),
        ],
        out_shape=[
            jax.ShapeDtypeStruct((NN, 128), jnp.float32),
            jax.ShapeDtypeStruct((NN, 16), jnp.float32),
            jax.ShapeDtypeStruct((NN, 16), jnp.float32),
        ],
    )(hp, rs, wo128, as2, ad2)


def _post_body(acc_ref, rs_ref, o_ref):
    comb = acc_ref[0][:, :NCLS] + acc_ref[1][:, :NCLS]
    rsum = rs_ref[0] + rs_ref[1] + 1e-9
    o = _elu(comb / rsum)
    m = jnp.max(o, axis=1, keepdims=True)
    lse = jnp.log(jnp.sum(jnp.exp(o - m), axis=1, keepdims=True))
    o_ref[...] = o - m - lse


def _post(acc2, rs2):
    return pl.pallas_call(
        _post_body,
        grid=(NN // NB,),
        in_specs=[
            pl.BlockSpec((2, NB, 128), lambda i: (0, i, 0)),
            pl.BlockSpec((2, NB, 1), lambda i: (0, i, 0)),
        ],
        out_specs=pl.BlockSpec((NB, NCLS), lambda i: (i, 0)),
        out_shape=jax.ShapeDtypeStruct((NN, NCLS), jnp.float32),
    )(acc2, rs2)


def _make_edge_pass(pair):
    ept = EE // NS if pair else EE // (NC * NS)
    nchunk = -(-ept // CHUNK)
    nj = 2 if pair else 1
    nscale = 8 if pair else 3
    shift = 6 if pair else 7
    nmask = 63 if pair else 127
    lmul = 2 if pair else 1
    nrs = 160 if pair else 80

    mesh = plsc.VectorSubcoreMesh(
        core_axis_name="c", subcore_axis_name="s",
        num_cores=NC, num_subcores=NS)

    @functools.partial(
        pl.kernel,
        out_type=[
            jax.ShapeDtypeStruct((NC, NN, 128), jnp.float32),
            jax.ShapeDtypeStruct((NC, nrs, 128), jnp.float32),
        ],
        mesh=mesh,
        compiler_params=pltpu.CompilerParams(needs_layout_passes=False),
        scratch_types=[
            pltpu.VMEM((CHUNK,), jnp.int32),
            pltpu.VMEM((CHUNK,), jnp.int32),
            pltpu.VMEM((CHUNK,), jnp.int32),
            pltpu.VMEM((CHUNK,), jnp.int32),
            pltpu.VMEM((CHUNK,), jnp.int32),
            pltpu.VMEM((CHUNK,), jnp.int32),
            pltpu.VMEM((CHUNK,), jnp.int32),
            pltpu.VMEM((CHUNK,), jnp.int32),
            pltpu.VMEM((CHUNK,), jnp.int32),
            pltpu.VMEM((NN,), jnp.int32 if pair else jnp.float32),
            pltpu.VMEM((NN,), jnp.int32 if pair else jnp.float32),
            pltpu.VMEM((CHUNK, 128), jnp.float32),
            pltpu.VMEM((CHUNK, 128), jnp.float32),
            pltpu.VMEM((CHUNK, 128), jnp.float32),
            pltpu.VMEM((CHUNK * 2 + L,), jnp.float32),
            pltpu.VMEM_SHARED((NN, 128), jnp.float32),
            pltpu.VMEM_SHARED((nrs, 128), jnp.float32),
            pltpu.SemaphoreType.DMA,
            pltpu.SemaphoreType.DMA,
            pltpu.SemaphoreType.DMA,
            pltpu.SemaphoreType.DMA,
            pltpu.SemaphoreType.DMA,
        ],
    )
    def edge_pass(adj, logits_hbm, tab_hbm,
                  acc_out, rs_out,
                  src_v0, src_v1, src_v2, dst_v0, dst_v1, dst_v2,
                  srow_v0, srow_v1, srow_v2, fsT, fdT,
                  t_b0, t_b1, rs_b, e_bf, acc, rsacc,
                  gsem0, gsem1, asem, rsem, isem):
        srcs = (src_v0, src_v1, src_v2)
        dsts = (dst_v0, dst_v1, dst_v2)
        srows = (srow_v0, srow_v1, srow_v2)
        t_bs = (t_b0, t_b1)
        gsems = (gsem0, gsem1)
        c = lax.axis_index("c")
        s = lax.axis_index("s")

        if pair:
            pltpu.sync_copy(logits_hbm.at[pl.ds(c * NN, NN)], fsT)
            pltpu.sync_copy(logits_hbm.at[pl.ds((2 + c) * NN, NN)], fdT)
        else:
            pltpu.sync_copy(logits_hbm.at[pl.ds(0, NN)], fsT)
            pltpu.sync_copy(logits_hbm.at[pl.ds(NN, NN)], fdT)

        def z_body(i, cr):
            for j in range(8):
                rs_b[i, pl.ds(j * L, L)] = jnp.zeros((L,), jnp.float32)
            return cr
        lax.fori_loop(0, CHUNK, z_body, 0)

        def zacc_body(i, cr):
            m = s + 16 * i

            @pl.when(m < NN // CHUNK)
            def _():
                pltpu.sync_copy(rs_b, acc.at[pl.ds(m * CHUNK, CHUNK)])
            return cr
        lax.fori_loop(0, (NN // CHUNK + 15) // 16, zacc_body, 0)
        if NN % CHUNK:
            @pl.when(s == NS - 1)
            def _init_acc_tail():
                pltpu.sync_copy(rs_b.at[pl.ds(0, NN % CHUNK)],
                                acc.at[pl.ds(NN - NN % CHUNK, NN % CHUNK)])

        @pl.when(s < nrs // 16)
        def _init_rs():
            pltpu.sync_copy(rs_b.at[pl.ds(0, 16)],
                            rsacc.at[pl.ds(s * 16, 16)])
        plsc.subcore_barrier()

        lane = lax.iota(jnp.int32, L)
        tile_e0 = s * ept if pair else c * (EE // NC) + s * ept
        zeros16 = jnp.zeros((L,), jnp.float32)
        zidx = jnp.zeros((L,), jnp.int32)
        bdnums = lax.GatherDimensionNumbers(
            offset_dims=(), collapsed_slice_dims=(0,), start_index_map=(0,))

        def bcast(v, iv):
            return lax.gather(v, iv[:, None], bdnums, (1,),
                              mode=lax.GatherScatterMode.PROMISE_IN_BOUNDS)

        def issue_idx(k, st):
            base = tile_e0 + k * CHUNK
            return (pltpu.async_copy(adj.at[pl.ds(base, CHUNK)],
                                     srcs[st], isem),
                    pltpu.async_copy(adj.at[pl.ds(EE + base, CHUNK)],
                                     dsts[st], isem))

        def prep_idx(st):
            for j in range(NG):
                sl = pl.ds(j * L, L)
                srows[st][sl] = lax.shift_right_logical(srcs[st][sl], shift)
            if pair:
                for j in range(NG):
                    sl = pl.ds(j * L, L)
                    dsts[st][sl] = dsts[st][sl] + c * NN

        def chunk_step(k, c2, c3):
            p2, p3, n3 = 1 - c2, (c3 + 2) % 3, (c3 + 1) % 3
            t_b = t_bs[c2]

            def e_body(g, cr):
                sidx = srcs[c3][pl.ds(g * L, L)]
                didx = dsts[c3][pl.ds(g * L, L)]
                if pair:
                    didx = didx - c * NN
                rowi = g * L + lane
                cbase = (sidx & nmask) * lmul
                ebase = rowi * 2
                ws = plsc.load_gather(fsT, [sidx])
                wd = plsc.load_gather(fdT, [didx])
                if pair:
                    hi = jnp.int32(-65536)
                    zs = (plsc.bitcast(jnp.left_shift(ws, 16), jnp.float32),
                          plsc.bitcast(jnp.bitwise_and(ws, hi), jnp.float32))
                    zd = (plsc.bitcast(jnp.left_shift(wd, 16), jnp.float32),
                          plsc.bitcast(jnp.bitwise_and(wd, hi), jnp.float32))
                else:
                    zs = (ws,)
                    zd = (wd,)
                inb = (k * CHUNK + rowi) < ept
                for j in range(nj):
                    z = zs[j] + zd[j]
                    ev = jnp.where(inb, jnp.exp(-jnp.maximum(z, z * ALPHA)),
                                   0.0)
                    plsc.store_scatter(rs_b, [rowi, cbase + j], ev)
                    plsc.store_scatter(e_bf, [ebase + j], ev)
                return cr
            for g_ in range(NG):
                e_body(g_, 0)

            rcp = pltpu.async_copy(rs_b, rsacc.at[srows[c3]], rsem, add=True)

            @pl.when(k > 0)
            def _():
                pltpu.make_async_copy(t_bs[p2], acc.at[srcs[p3]],
                                      asem).wait()

            @pl.when(k + 1 < nchunk)
            def _():
                pltpu.async_copy(tab_hbm.at[dsts[n3]], t_bs[p2],
                                 gsems[p2])

            @pl.when(k + 2 < nchunk)
            def _():
                issue_idx(k + 2, p3)

            pltpu.make_async_copy(tab_hbm.at[dsts[c3]], t_b,
                                  gsems[c2]).wait()

            def s_body(i, cr):
                ev = e_bf[pl.ds(i * 2, L)]
                eA = bcast(ev, zidx)
                eB = bcast(ev, zidx + 1) if pair else eA
                for j in range(nscale):
                    sl = pl.ds(j * L, L)
                    ee = eA if (not pair or j < 4) else eB
                    t_b[i, sl] = t_b[i, sl] * ee
                return cr
            lax.fori_loop(0, CHUNK, s_body, 0, unroll=8)

            pltpu.async_copy(t_b, acc.at[srcs[c3]], asem, add=True)

            rcp.wait()

            def uz_body(g, cr):
                sidx = srcs[c3][pl.ds(g * L, L)]
                rowi = g * L + lane
                cbase = (sidx & nmask) * lmul
                for j in range(nj):
                    plsc.store_scatter(rs_b, [rowi, cbase + j], zeros16)
                return cr
            for g_ in range(NG):
                uz_body(g_, 0)

            @pl.when(k + 2 < nchunk)
            def _():
                base = tile_e0 + (k + 2) * CHUNK
                pltpu.make_async_copy(adj.at[pl.ds(base, CHUNK)],
                                      srcs[p3], isem).wait()
                pltpu.make_async_copy(adj.at[pl.ds(EE + base, CHUNK)],
                                      dsts[p3], isem).wait()
                prep_idx(p3)

        cps = issue_idx(jnp.int32(0), 0) + issue_idx(jnp.int32(1), 1)
        for cp_ in cps:
            cp_.wait()
        prep_idx(0)
        prep_idx(1)
        pltpu.async_copy(tab_hbm.at[dsts[0]], t_bs[0], gsems[0])

        assert nchunk % 6 == 1
        def six_body(i, carry):
            for b in range(6):
                chunk_step(6 * i + b, b % 2, b % 3)
            return carry
        lax.fori_loop(0, nchunk // 6, six_body, 0)
        chunk_step(jnp.int32(nchunk - 1), 0, 0)
        pltpu.make_async_copy(t_bs[0], acc.at[srcs[0]], asem).wait()

        plsc.subcore_barrier()

        @pl.when(s < NTD)
        def _drain_acc():
            sl = pl.ds(s * RPT, RPT)
            pltpu.sync_copy(acc.at[sl], acc_out.at[c, sl])

        @pl.when(s == 0)
        def _drain_rs():
            pltpu.sync_copy(rsacc, rs_out.at[c])

    return edge_pass


_edge_pass1 = _make_edge_pass(True)
_edge_pass2 = _make_edge_pass(False)


def kernel(x, adj, W0, W1, W2, W3, a0, a1, a2, a3, Wout, aout):
    f32 = jnp.float32
    wcat = jnp.concatenate([W0, W1, W2, W3], axis=1)
    asrc = jnp.zeros((256, 16), f32)
    adst = jnp.zeros((256, 16), f32)
    for h, a in enumerate([a0, a1, a2, a3]):
        asrc = asrc.at[h * NHID:(h + 1) * NHID, h].set(a[:NHID])
        adst = adst.at[h * NHID:(h + 1) * NHID, h].set(a[NHID:])
    wo128 = jnp.zeros((256, 128), f32).at[:, :NCLS].set(Wout)
    as2 = jnp.zeros((128, 16), f32).at[:NCLS, 0].set(aout[:NCLS])
    ad2 = jnp.zeros((128, 16), f32).at[:NCLS, 0].set(aout[NCLS:])

    hflat, fs16, fd16 = _pre(x, wcat, asrc, adst)

    def pack2(a, b):
        ab = lax.bitcast_convert_type(a.astype(jnp.bfloat16), jnp.uint16)
        bb = lax.bitcast_convert_type(b.astype(jnp.bfloat16), jnp.uint16)
        w = ab.astype(jnp.uint32) | (bb.astype(jnp.uint32) << 16)
        return lax.bitcast_convert_type(w, jnp.int32)

    logits1 = jnp.concatenate(
        [pack2(fs16[:, 0], fs16[:, 1]), pack2(fs16[:, 2], fs16[:, 3]),
         pack2(fd16[:, 0], fd16[:, 1]), pack2(fd16[:, 2], fd16[:, 3])])
    adjf = jnp.concatenate(
        [adj.reshape(2 * EE), jnp.zeros((CHUNK,), jnp.int32)])
    hp, rs1 = _edge_pass1(adjf, logits1, hflat.reshape(2 * NN, 128))
    rs4 = rs1.reshape(NC, 160 * 64, 2)[:, :NN, :].transpose(1, 0, 2).reshape(NN, 4)
    g128, gs16, gd16 = _mid(hp, rs4, wo128, as2, ad2)
    logits2 = jnp.concatenate([gs16[:, 0], gd16[:, 0]])
    acc2, rs2 = _edge_pass2(adjf, logits2, g128)
    rs2u = rs2.reshape(NC, 80 * 128)[:, :NN].reshape(NC, NN, 1)
    out = _post(acc2, rs2u)
    return out


# speedup vs baseline: 1.3911x; 1.0277x over previous
import functools

import jax
import jax.numpy as jnp
from jax import lax
from jax.experimental import pallas as pl
from jax.experimental.pallas import tpu as pltpu
from jax.experimental.pallas import tpu_sc as plsc

NN = 10000
EE = 320000
NFEAT = 128
NHID = 64
NCLS = 40
ALPHA = 0.2
NC, NS, L = 2, 16, 16
CHUNK = 64
NG = CHUNK // L
RPT = 1000
NTD = NN // RPT
NB = 1000


def _elu(v):
    return jnp.where(v > 0, v, jnp.exp(jnp.minimum(v, 0.0)) - 1.0)


def _pre_body(x_ref, wc_ref, as_ref, ad_ref, hf_ref, fs_ref, fd_ref):
    h = jnp.dot(x_ref[...], wc_ref[...], preferred_element_type=jnp.float32)
    hf_ref[0] = h[:, :128]
    hf_ref[1] = h[:, 128:]
    fs_ref[...] = jnp.dot(h, as_ref[...], preferred_element_type=jnp.float32)
    fd_ref[...] = jnp.dot(h, ad_ref[...], preferred_element_type=jnp.float32)


def _pre(x, wcat, asrc, adst):
    return pl.pallas_call(
        _pre_body,
        grid=(NN // NB,),
        in_specs=[
            pl.BlockSpec((NB, NFEAT), lambda i: (i, 0)),
            pl.BlockSpec((NFEAT, 256), lambda i: (0, 0)),
            pl.BlockSpec((256, 16), lambda i: (0, 0)),
            pl.BlockSpec((256, 16), lambda i: (0, 0)),
        ],
        out_specs=[
            pl.BlockSpec((2, NB, 128), lambda i: (0, i, 0)),
            pl.BlockSpec((NB, 16), lambda i: (i, 0)),
            pl.BlockSpec((NB, 16), lambda i: (i, 0)),
        ],
        out_shape=[
            jax.ShapeDtypeStruct((2, NN, 128), jnp.float32),
            jax.ShapeDtypeStruct((NN, 16), jnp.float32),
            jax.ShapeDtypeStruct((NN, 16), jnp.float32),
        ],
    )(x, wcat, asrc, adst)


def _mid_body(hp_ref, rs_ref, wo_ref, as2_ref, ad2_ref, g_ref, gs_ref, gd_ref):
    parts = []
    for head in range(4):
        c, j = divmod(head, 2)
        hp = hp_ref[c][:, j * NHID:(j + 1) * NHID]
        denom = rs_ref[:, head][:, None] + 1e-9
        parts.append(_elu(hp / denom))
    x1 = jnp.concatenate(parts, axis=1)
    g = jnp.dot(x1, wo_ref[...], preferred_element_type=jnp.float32)
    col = lax.broadcasted_iota(jnp.int32, g.shape, 1)
    g_ref[...] = jnp.where(col == NCLS + 8, 1.0, g)
    gs_ref[...] = jnp.dot(g, as2_ref[...], preferred_element_type=jnp.float32)
    gd_ref[...] = jnp.dot(g, ad2_ref[...], preferred_element_type=jnp.float32)


def _mid(hp, rs, wo128, as2, ad2):
    return pl.pallas_call(
        _mid_body,
        grid=(NN // NB,),
        in_specs=[
            pl.BlockSpec((2, NB, 128), lambda i: (0, i, 0)),
            pl.BlockSpec((NB, 4), lambda i: (i, 0)),
            pl.BlockSpec((256, 128), lambda i: (0, 0)),
            pl.BlockSpec((128, 16), lambda i: (0, 0)),
            pl.BlockSpec((128, 16), lambda i: (0, 0)),
        ],
        out_specs=[
            pl.BlockSpec((NB, 128), lambda i: (i, 0)),
            pl.BlockSpec((NB, 16), lambda i: (i, 0)),
            pl.BlockSpec((NB, 16), lambda i: (i, 0)),
        ],
        out_shape=[
            jax.ShapeDtypeStruct((NN, 128), jnp.float32),
            jax.ShapeDtypeStruct((NN, 16), jnp.float32),
            jax.ShapeDtypeStruct((NN, 16), jnp.float32),
        ],
    )(hp, rs, wo128, as2, ad2)


def _post_body(acc_ref, o_ref):
    comb = acc_ref[0][:, :NCLS] + acc_ref[1][:, :NCLS]
    rsum = (acc_ref[0][:, NCLS + 8] + acc_ref[1][:, NCLS + 8])[:, None] + 1e-9
    o = _elu(comb / rsum)
    m = jnp.max(o, axis=1, keepdims=True)
    lse = jnp.log(jnp.sum(jnp.exp(o - m), axis=1, keepdims=True))
    o_ref[...] = o - m - lse


def _post(acc2):
    return pl.pallas_call(
        _post_body,
        grid=(NN // NB,),
        in_specs=[
            pl.BlockSpec((2, NB, 128), lambda i: (0, i, 0)),
        ],
        out_specs=pl.BlockSpec((NB, NCLS), lambda i: (i, 0)),
        out_shape=jax.ShapeDtypeStruct((NN, NCLS), jnp.float32),
    )(acc2)


def _make_edge_pass(pair):
    ept = EE // NS if pair else EE // (NC * NS)
    nchunk = -(-ept // CHUNK)
    nj = 2 if pair else 1
    nscale = 8 if pair else 4
    shift = 6 if pair else 7
    nmask = 63 if pair else 127
    lmul = 2 if pair else 1
    nrs = 160 if pair else 80

    mesh = plsc.VectorSubcoreMesh(
        core_axis_name="c", subcore_axis_name="s",
        num_cores=NC, num_subcores=NS)

    @functools.partial(
        pl.kernel,
        out_type=[
            jax.ShapeDtypeStruct((NC, NN, 128), jnp.float32),
            jax.ShapeDtypeStruct((NC, nrs, 128), jnp.float32),
        ],
        mesh=mesh,
        compiler_params=pltpu.CompilerParams(needs_layout_passes=False),
        scratch_types=[
            pltpu.VMEM((CHUNK,), jnp.int32),
            pltpu.VMEM((CHUNK,), jnp.int32),
            pltpu.VMEM((CHUNK,), jnp.int32),
            pltpu.VMEM((CHUNK,), jnp.int32),
            pltpu.VMEM((CHUNK,), jnp.int32),
            pltpu.VMEM((CHUNK,), jnp.int32),
            pltpu.VMEM((CHUNK,), jnp.int32),
            pltpu.VMEM((CHUNK,), jnp.int32),
            pltpu.VMEM((CHUNK,), jnp.int32),
            pltpu.VMEM((NN,), jnp.int32 if pair else jnp.float32),
            pltpu.VMEM((NN,), jnp.int32 if pair else jnp.float32),
            pltpu.VMEM((CHUNK, 128), jnp.float32),
            pltpu.VMEM((CHUNK, 128), jnp.float32),
            pltpu.VMEM((CHUNK, 128), jnp.float32),
            pltpu.VMEM((CHUNK * 2 + L,), jnp.float32),
            pltpu.VMEM_SHARED((NN, 128), jnp.float32),
            pltpu.VMEM_SHARED((nrs, 128), jnp.float32),
            pltpu.SemaphoreType.DMA,
            pltpu.SemaphoreType.DMA,
            pltpu.SemaphoreType.DMA,
            pltpu.SemaphoreType.DMA,
            pltpu.SemaphoreType.DMA,
        ],
    )
    def edge_pass(adj, logits_hbm, tab_hbm,
                  acc_out, rs_out,
                  src_v0, src_v1, src_v2, dst_v0, dst_v1, dst_v2,
                  srow_v0, srow_v1, srow_v2, fsT, fdT,
                  t_b0, t_b1, rs_b, e_bf, acc, rsacc,
                  gsem0, gsem1, asem, rsem, isem):
        srcs = (src_v0, src_v1, src_v2)
        dsts = (dst_v0, dst_v1, dst_v2)
        srows = (srow_v0, srow_v1, srow_v2)
        t_bs = (t_b0, t_b1)
        gsems = (gsem0, gsem1)
        c = lax.axis_index("c")
        s = lax.axis_index("s")

        if pair:
            pltpu.sync_copy(logits_hbm.at[pl.ds(c * NN, NN)], fsT)
            pltpu.sync_copy(logits_hbm.at[pl.ds((2 + c) * NN, NN)], fdT)
        else:
            pltpu.sync_copy(logits_hbm.at[pl.ds(0, NN)], fsT)
            pltpu.sync_copy(logits_hbm.at[pl.ds(NN, NN)], fdT)

        def z_body(i, cr):
            for j in range(8):
                rs_b[i, pl.ds(j * L, L)] = jnp.zeros((L,), jnp.float32)
            return cr
        lax.fori_loop(0, CHUNK, z_body, 0)

        def zacc_body(i, cr):
            m = s + 16 * i

            @pl.when(m < NN // CHUNK)
            def _():
                pltpu.sync_copy(rs_b, acc.at[pl.ds(m * CHUNK, CHUNK)])
            return cr
        lax.fori_loop(0, (NN // CHUNK + 15) // 16, zacc_body, 0)
        if NN % CHUNK:
            @pl.when(s == NS - 1)
            def _init_acc_tail():
                pltpu.sync_copy(rs_b.at[pl.ds(0, NN % CHUNK)],
                                acc.at[pl.ds(NN - NN % CHUNK, NN % CHUNK)])

        if pair:
            @pl.when(s < nrs // 16)
            def _init_rs():
                pltpu.sync_copy(rs_b.at[pl.ds(0, 16)],
                                rsacc.at[pl.ds(s * 16, 16)])
        plsc.subcore_barrier()

        lane = lax.iota(jnp.int32, L)
        tile_e0 = s * ept if pair else c * (EE // NC) + s * ept
        zeros16 = jnp.zeros((L,), jnp.float32)
        zidx = jnp.zeros((L,), jnp.int32)
        bdnums = lax.GatherDimensionNumbers(
            offset_dims=(), collapsed_slice_dims=(0,), start_index_map=(0,))

        def bcast(v, iv):
            return lax.gather(v, iv[:, None], bdnums, (1,),
                              mode=lax.GatherScatterMode.PROMISE_IN_BOUNDS)

        def issue_idx(k, st):
            base = tile_e0 + k * CHUNK
            return (pltpu.async_copy(adj.at[pl.ds(base, CHUNK)],
                                     srcs[st], isem),
                    pltpu.async_copy(adj.at[pl.ds(EE + base, CHUNK)],
                                     dsts[st], isem))

        def prep_idx(st):
            for j in range(NG):
                sl = pl.ds(j * L, L)
                srows[st][sl] = lax.shift_right_logical(srcs[st][sl], shift)
            if pair:
                for j in range(NG):
                    sl = pl.ds(j * L, L)
                    dsts[st][sl] = dsts[st][sl] + c * NN

        def chunk_step(k, c2, c3):
            p2, p3, n3 = 1 - c2, (c3 + 2) % 3, (c3 + 1) % 3
            t_b = t_bs[c2]

            def e_body(g, cr):
                sidx = srcs[c3][pl.ds(g * L, L)]
                didx = dsts[c3][pl.ds(g * L, L)]
                if pair:
                    didx = didx - c * NN
                rowi = g * L + lane
                cbase = (sidx & nmask) * lmul
                ebase = rowi * 2
                ws = plsc.load_gather(fsT, [sidx])
                wd = plsc.load_gather(fdT, [didx])
                if pair:
                    hi = jnp.int32(-65536)
                    zs = (plsc.bitcast(jnp.left_shift(ws, 16), jnp.float32),
                          plsc.bitcast(jnp.bitwise_and(ws, hi), jnp.float32))
                    zd = (plsc.bitcast(jnp.left_shift(wd, 16), jnp.float32),
                          plsc.bitcast(jnp.bitwise_and(wd, hi), jnp.float32))
                else:
                    zs = (ws,)
                    zd = (wd,)
                inb = (k * CHUNK + rowi) < ept
                for j in range(nj):
                    z = zs[j] + zd[j]
                    ev = jnp.where(inb, jnp.exp(-jnp.maximum(z, z * ALPHA)),
                                   0.0)
                    if pair:
                        plsc.store_scatter(rs_b, [rowi, cbase + j], ev)
                    plsc.store_scatter(e_bf, [ebase + j], ev)
                return cr
            for g_ in range(NG):
                e_body(g_, 0)

            if pair:
                rcp = pltpu.async_copy(rs_b, rsacc.at[srows[c3]], rsem,
                                       add=True)

            @pl.when(k > 0)
            def _():
                pltpu.make_async_copy(t_bs[p2], acc.at[srcs[p3]],
                                      asem).wait()

            @pl.when(k + 1 < nchunk)
            def _():
                pltpu.async_copy(tab_hbm.at[dsts[n3]], t_bs[p2],
                                 gsems[p2])

            @pl.when(k + 2 < nchunk)
            def _():
                issue_idx(k + 2, p3)

            pltpu.make_async_copy(tab_hbm.at[dsts[c3]], t_b,
                                  gsems[c2]).wait()

            def s_body(i, cr):
                ev = e_bf[pl.ds(i * 2, L)]
                eA = bcast(ev, zidx)
                eB = bcast(ev, zidx + 1) if pair else eA
                for j in range(nscale):
                    sl = pl.ds(j * L, L)
                    ee = eA if (not pair or j < 4) else eB
                    t_b[i, sl] = t_b[i, sl] * ee
                return cr
            lax.fori_loop(0, CHUNK, s_body, 0, unroll=8)

            pltpu.async_copy(t_b, acc.at[srcs[c3]], asem, add=True)

            if pair:
                rcp.wait()

                def uz_body(g, cr):
                    sidx = srcs[c3][pl.ds(g * L, L)]
                    rowi = g * L + lane
                    cbase = (sidx & nmask) * lmul
                    for j in range(nj):
                        plsc.store_scatter(rs_b, [rowi, cbase + j], zeros16)
                    return cr
                for g_ in range(NG):
                    uz_body(g_, 0)

            @pl.when(k + 2 < nchunk)
            def _():
                base = tile_e0 + (k + 2) * CHUNK
                pltpu.make_async_copy(adj.at[pl.ds(base, CHUNK)],
                                      srcs[p3], isem).wait()
                pltpu.make_async_copy(adj.at[pl.ds(EE + base, CHUNK)],
                                      dsts[p3], isem).wait()
                prep_idx(p3)

        cps = issue_idx(jnp.int32(0), 0) + issue_idx(jnp.int32(1), 1)
        for cp_ in cps:
            cp_.wait()
        prep_idx(0)
        prep_idx(1)
        pltpu.async_copy(tab_hbm.at[dsts[0]], t_bs[0], gsems[0])

        assert nchunk % 6 == 1
        def six_body(i, carry):
            for b in range(6):
                chunk_step(6 * i + b, b % 2, b % 3)
            return carry
        lax.fori_loop(0, nchunk // 6, six_body, 0)
        chunk_step(jnp.int32(nchunk - 1), 0, 0)
        pltpu.make_async_copy(t_bs[0], acc.at[srcs[0]], asem).wait()

        plsc.subcore_barrier()

        @pl.when(s < NTD)
        def _drain_acc():
            sl = pl.ds(s * RPT, RPT)
            pltpu.sync_copy(acc.at[sl], acc_out.at[c, sl])

        if pair:
            @pl.when(s == 0)
            def _drain_rs():
                pltpu.sync_copy(rsacc, rs_out.at[c])

    return edge_pass


_edge_pass1 = _make_edge_pass(True)
_edge_pass2 = _make_edge_pass(False)


def kernel(x, adj, W0, W1, W2, W3, a0, a1, a2, a3, Wout, aout):
    f32 = jnp.float32
    wcat = jnp.concatenate([W0, W1, W2, W3], axis=1)
    asrc = jnp.zeros((256, 16), f32)
    adst = jnp.zeros((256, 16), f32)
    for h, a in enumerate([a0, a1, a2, a3]):
        asrc = asrc.at[h * NHID:(h + 1) * NHID, h].set(a[:NHID])
        adst = adst.at[h * NHID:(h + 1) * NHID, h].set(a[NHID:])
    wo128 = jnp.zeros((256, 128), f32).at[:, :NCLS].set(Wout)
    as2 = jnp.zeros((128, 16), f32).at[:NCLS, 0].set(aout[:NCLS])
    ad2 = jnp.zeros((128, 16), f32).at[:NCLS, 0].set(aout[NCLS:])

    hflat, fs16, fd16 = _pre(x, wcat, asrc, adst)

    def pack2(a, b):
        ab = lax.bitcast_convert_type(a.astype(jnp.bfloat16), jnp.uint16)
        bb = lax.bitcast_convert_type(b.astype(jnp.bfloat16), jnp.uint16)
        w = ab.astype(jnp.uint32) | (bb.astype(jnp.uint32) << 16)
        return lax.bitcast_convert_type(w, jnp.int32)

    logits1 = jnp.concatenate(
        [pack2(fs16[:, 0], fs16[:, 1]), pack2(fs16[:, 2], fs16[:, 3]),
         pack2(fd16[:, 0], fd16[:, 1]), pack2(fd16[:, 2], fd16[:, 3])])
    adjf = jnp.concatenate(
        [adj.reshape(2 * EE), jnp.zeros((CHUNK,), jnp.int32)])
    hp, rs1 = _edge_pass1(adjf, logits1, hflat.reshape(2 * NN, 128))
    rs4 = rs1.reshape(NC, 160 * 64, 2)[:, :NN, :].transpose(1, 0, 2).reshape(NN, 4)
    g128, gs16, gd16 = _mid(hp, rs4, wo128, as2, ad2)
    logits2 = jnp.concatenate([gs16[:, 0], gd16[:, 0]])
    acc2, _ = _edge_pass2(adjf, logits2, g128)
    return _post(acc2)


# speedup vs baseline: 1.4476x; 1.0406x over previous
import functools

import jax
import jax.numpy as jnp
from jax import lax
from jax.experimental import pallas as pl
from jax.experimental.pallas import tpu as pltpu
from jax.experimental.pallas import tpu_sc as plsc

NN = 10000
EE = 320000
NFEAT = 128
NHID = 64
NCLS = 40
ALPHA = 0.2
NC, NS, L = 2, 16, 16
CHUNK = 64
NG = CHUNK // L
RPT = 1000
NTD = NN // RPT
NB = 1000


def _elu(v):
    return jnp.where(v > 0, v, jnp.exp(jnp.minimum(v, 0.0)) - 1.0)


def _pre_body(x_ref, wc_ref, as_ref, ad_ref, hf_ref, fs_ref, fd_ref):
    h = jnp.dot(x_ref[...], wc_ref[...], preferred_element_type=jnp.float32)
    hf_ref[0] = h[:, :128]
    hf_ref[1] = h[:, 128:]
    fs_ref[...] = jnp.dot(h, as_ref[...], preferred_element_type=jnp.float32)
    fd_ref[...] = jnp.dot(h, ad_ref[...], preferred_element_type=jnp.float32)


def _pre(x, wcat, asrc, adst):
    return pl.pallas_call(
        _pre_body,
        grid=(NN // NB,),
        in_specs=[
            pl.BlockSpec((NB, NFEAT), lambda i: (i, 0)),
            pl.BlockSpec((NFEAT, 256), lambda i: (0, 0)),
            pl.BlockSpec((256, 16), lambda i: (0, 0)),
            pl.BlockSpec((256, 16), lambda i: (0, 0)),
        ],
        out_specs=[
            pl.BlockSpec((2, NB, 128), lambda i: (0, i, 0)),
            pl.BlockSpec((NB, 16), lambda i: (i, 0)),
            pl.BlockSpec((NB, 16), lambda i: (i, 0)),
        ],
        out_shape=[
            jax.ShapeDtypeStruct((2, NN, 128), jnp.float32),
            jax.ShapeDtypeStruct((NN, 16), jnp.float32),
            jax.ShapeDtypeStruct((NN, 16), jnp.float32),
        ],
    )(x, wcat, asrc, adst)


def _mid_body(hp_ref, rs_ref, wo_ref, as2_ref, ad2_ref, g_ref, gs_ref, gd_ref):
    parts = []
    for head in range(4):
        c, j = divmod(head, 2)
        hp = hp_ref[c][:, j * NHID:(j + 1) * NHID]
        denom = rs_ref[:, head][:, None] + 1e-9
        parts.append(_elu(hp / denom))
    x1 = jnp.concatenate(parts, axis=1)
    g = jnp.dot(x1, wo_ref[...], preferred_element_type=jnp.float32)
    col = lax.broadcasted_iota(jnp.int32, g.shape, 1)
    g_ref[...] = jnp.where(col == NCLS + 8, 1.0, g)
    gs_ref[...] = jnp.dot(g, as2_ref[...], preferred_element_type=jnp.float32)
    gd_ref[...] = jnp.dot(g, ad2_ref[...], preferred_element_type=jnp.float32)


def _mid(hp, rs, wo128, as2, ad2):
    return pl.pallas_call(
        _mid_body,
        grid=(NN // NB,),
        in_specs=[
            pl.BlockSpec((2, NB, 128), lambda i: (0, i, 0)),
            pl.BlockSpec((NB, 4), lambda i: (i, 0)),
            pl.BlockSpec((256, 64), lambda i: (0, 0)),
            pl.BlockSpec((64, 16), lambda i: (0, 0)),
            pl.BlockSpec((64, 16), lambda i: (0, 0)),
        ],
        out_specs=[
            pl.BlockSpec((NB, 64), lambda i: (i, 0)),
            pl.BlockSpec((NB, 16), lambda i: (i, 0)),
            pl.BlockSpec((NB, 16), lambda i: (i, 0)),
        ],
        out_shape=[
            jax.ShapeDtypeStruct((NN, 64), jnp.float32),
            jax.ShapeDtypeStruct((NN, 16), jnp.float32),
            jax.ShapeDtypeStruct((NN, 16), jnp.float32),
        ],
    )(hp, rs, wo128, as2, ad2)


def _post_body(acc_ref, o_ref):
    comb = acc_ref[0][:, :NCLS] + acc_ref[1][:, :NCLS]
    rsum = (acc_ref[0][:, NCLS + 8] + acc_ref[1][:, NCLS + 8])[:, None] + 1e-9
    o = _elu(comb / rsum)
    m = jnp.max(o, axis=1, keepdims=True)
    lse = jnp.log(jnp.sum(jnp.exp(o - m), axis=1, keepdims=True))
    o_ref[...] = o - m - lse


def _post(acc2):
    return pl.pallas_call(
        _post_body,
        grid=(NN // NB,),
        in_specs=[
            pl.BlockSpec((2, NB, 64), lambda i: (0, i, 0)),
        ],
        out_specs=pl.BlockSpec((NB, NCLS), lambda i: (i, 0)),
        out_shape=jax.ShapeDtypeStruct((NN, NCLS), jnp.float32),
    )(acc2)


def _make_edge_pass(pair):
    ept = EE // NS if pair else EE // (NC * NS)
    nchunk = -(-ept // CHUNK)
    nj = 2 if pair else 1
    tw = 128 if pair else 64
    nscale = 8 if pair else 4
    shift = 6 if pair else 7
    nmask = 63 if pair else 127
    lmul = 2 if pair else 1
    nrs = 160 if pair else 80

    mesh = plsc.VectorSubcoreMesh(
        core_axis_name="c", subcore_axis_name="s",
        num_cores=NC, num_subcores=NS)

    @functools.partial(
        pl.kernel,
        out_type=[
            jax.ShapeDtypeStruct((NC, NN, tw), jnp.float32),
            jax.ShapeDtypeStruct((NC, nrs, 128), jnp.float32),
        ],
        mesh=mesh,
        compiler_params=pltpu.CompilerParams(
            needs_layout_passes=False,
            use_tc_tiling_on_sc=pair),
        scratch_types=[
            pltpu.VMEM((CHUNK,), jnp.int32),
            pltpu.VMEM((CHUNK,), jnp.int32),
            pltpu.VMEM((CHUNK,), jnp.int32),
            pltpu.VMEM((CHUNK,), jnp.int32),
            pltpu.VMEM((CHUNK,), jnp.int32),
            pltpu.VMEM((CHUNK,), jnp.int32),
            pltpu.VMEM((CHUNK,), jnp.int32),
            pltpu.VMEM((CHUNK,), jnp.int32),
            pltpu.VMEM((CHUNK,), jnp.int32),
            pltpu.VMEM((NN,), jnp.int32 if pair else jnp.float32),
            pltpu.VMEM((NN,), jnp.int32 if pair else jnp.float32),
            pltpu.VMEM((CHUNK, tw), jnp.float32),
            pltpu.VMEM((CHUNK, tw), jnp.float32),
            pltpu.VMEM((CHUNK, 128), jnp.float32),
            pltpu.VMEM((CHUNK * 2 + L,), jnp.float32),
            pltpu.VMEM_SHARED((NN, tw), jnp.float32),
            pltpu.VMEM_SHARED((nrs, 128), jnp.float32),
            pltpu.SemaphoreType.DMA,
            pltpu.SemaphoreType.DMA,
            pltpu.SemaphoreType.DMA,
            pltpu.SemaphoreType.DMA,
            pltpu.SemaphoreType.DMA,
        ],
    )
    def edge_pass(adj, logits_hbm, tab_hbm,
                  acc_out, rs_out,
                  src_v0, src_v1, src_v2, dst_v0, dst_v1, dst_v2,
                  srow_v0, srow_v1, srow_v2, fsT, fdT,
                  t_b0, t_b1, rs_b, e_bf, acc, rsacc,
                  gsem0, gsem1, asem, rsem, isem):
        srcs = (src_v0, src_v1, src_v2)
        dsts = (dst_v0, dst_v1, dst_v2)
        srows = (srow_v0, srow_v1, srow_v2)
        t_bs = (t_b0, t_b1)
        gsems = (gsem0, gsem1)
        c = lax.axis_index("c")
        s = lax.axis_index("s")

        if pair:
            pltpu.sync_copy(logits_hbm.at[pl.ds(c * NN, NN)], fsT)
            pltpu.sync_copy(logits_hbm.at[pl.ds((2 + c) * NN, NN)], fdT)
        else:
            pltpu.sync_copy(logits_hbm.at[pl.ds(0, NN)], fsT)
            pltpu.sync_copy(logits_hbm.at[pl.ds(NN, NN)], fdT)

        zsrc = rs_b if pair else t_b0

        def z_body(i, cr):
            for j in range((128 if pair else tw) // L):
                zsrc[i, pl.ds(j * L, L)] = jnp.zeros((L,), jnp.float32)
            return cr
        lax.fori_loop(0, CHUNK, z_body, 0)

        def zacc_body(i, cr):
            m = s + 16 * i

            @pl.when(m < NN // CHUNK)
            def _():
                pltpu.sync_copy(zsrc, acc.at[pl.ds(m * CHUNK, CHUNK)])
            return cr
        lax.fori_loop(0, (NN // CHUNK + 15) // 16, zacc_body, 0)
        if NN % CHUNK:
            @pl.when(s == NS - 1)
            def _init_acc_tail():
                pltpu.sync_copy(zsrc.at[pl.ds(0, NN % CHUNK)],
                                acc.at[pl.ds(NN - NN % CHUNK, NN % CHUNK)])

        if pair:
            @pl.when(s < nrs // 16)
            def _init_rs():
                pltpu.sync_copy(rs_b.at[pl.ds(0, 16)],
                                rsacc.at[pl.ds(s * 16, 16)])
        plsc.subcore_barrier()

        lane = lax.iota(jnp.int32, L)
        tile_e0 = s * ept if pair else c * (EE // NC) + s * ept
        zeros16 = jnp.zeros((L,), jnp.float32)
        zidx = jnp.zeros((L,), jnp.int32)
        bdnums = lax.GatherDimensionNumbers(
            offset_dims=(), collapsed_slice_dims=(0,), start_index_map=(0,))

        def bcast(v, iv):
            return lax.gather(v, iv[:, None], bdnums, (1,),
                              mode=lax.GatherScatterMode.PROMISE_IN_BOUNDS)

        def issue_idx(k, st):
            base = tile_e0 + k * CHUNK
            return (pltpu.async_copy(adj.at[pl.ds(base, CHUNK)],
                                     srcs[st], isem),
                    pltpu.async_copy(adj.at[pl.ds(EE + base, CHUNK)],
                                     dsts[st], isem))

        def prep_idx(st):
            for j in range(NG):
                sl = pl.ds(j * L, L)
                srows[st][sl] = lax.shift_right_logical(srcs[st][sl], shift)
            if pair:
                for j in range(NG):
                    sl = pl.ds(j * L, L)
                    dsts[st][sl] = dsts[st][sl] + c * NN

        def chunk_step(k, c2, c3):
            p2, p3, n3 = 1 - c2, (c3 + 2) % 3, (c3 + 1) % 3
            t_b = t_bs[c2]

            def e_body(g, cr):
                sidx = srcs[c3][pl.ds(g * L, L)]
                didx = dsts[c3][pl.ds(g * L, L)]
                if pair:
                    didx = didx - c * NN
                rowi = g * L + lane
                cbase = (sidx & nmask) * lmul
                ebase = rowi * 2
                ws = plsc.load_gather(fsT, [sidx])
                wd = plsc.load_gather(fdT, [didx])
                if pair:
                    hi = jnp.int32(-65536)
                    zs = (plsc.bitcast(jnp.left_shift(ws, 16), jnp.float32),
                          plsc.bitcast(jnp.bitwise_and(ws, hi), jnp.float32))
                    zd = (plsc.bitcast(jnp.left_shift(wd, 16), jnp.float32),
                          plsc.bitcast(jnp.bitwise_and(wd, hi), jnp.float32))
                else:
                    zs = (ws,)
                    zd = (wd,)
                inb = (k * CHUNK + rowi) < ept
                for j in range(nj):
                    z = zs[j] + zd[j]
                    ev = jnp.where(inb, jnp.exp(-jnp.maximum(z, z * ALPHA)),
                                   0.0)
                    if pair:
                        plsc.store_scatter(rs_b, [rowi, cbase + j], ev)
                    plsc.store_scatter(e_bf, [ebase + j], ev)
                return cr
            for g_ in range(NG):
                e_body(g_, 0)

            if pair:
                rcp = pltpu.async_copy(rs_b, rsacc.at[srows[c3]], rsem,
                                       add=True)

            @pl.when(k > 0)
            def _():
                pltpu.make_async_copy(t_bs[p2], acc.at[srcs[p3]],
                                      asem).wait()

            @pl.when(k + 1 < nchunk)
            def _():
                pltpu.async_copy(tab_hbm.at[dsts[n3]], t_bs[p2],
                                 gsems[p2])

            @pl.when(k + 2 < nchunk)
            def _():
                issue_idx(k + 2, p3)

            pltpu.make_async_copy(tab_hbm.at[dsts[c3]], t_b,
                                  gsems[c2]).wait()

            def s_body(i, cr):
                ev = e_bf[pl.ds(i * 2, L)]
                eA = bcast(ev, zidx)
                eB = bcast(ev, zidx + 1) if pair else eA
                for j in range(nscale):
                    sl = pl.ds(j * L, L)
                    ee = eA if (not pair or j < 4) else eB
                    t_b[i, sl] = t_b[i, sl] * ee
                return cr
            lax.fori_loop(0, CHUNK, s_body, 0, unroll=8)

            pltpu.async_copy(t_b, acc.at[srcs[c3]], asem, add=True)

            if pair:
                rcp.wait()

                def uz_body(g, cr):
                    sidx = srcs[c3][pl.ds(g * L, L)]
                    rowi = g * L + lane
                    cbase = (sidx & nmask) * lmul
                    for j in range(nj):
                        plsc.store_scatter(rs_b, [rowi, cbase + j], zeros16)
                    return cr
                for g_ in range(NG):
                    uz_body(g_, 0)

            @pl.when(k + 2 < nchunk)
            def _():
                base = tile_e0 + (k + 2) * CHUNK
                pltpu.make_async_copy(adj.at[pl.ds(base, CHUNK)],
                                      srcs[p3], isem).wait()
                pltpu.make_async_copy(adj.at[pl.ds(EE + base, CHUNK)],
                                      dsts[p3], isem).wait()
                prep_idx(p3)

        cps = issue_idx(jnp.int32(0), 0) + issue_idx(jnp.int32(1), 1)
        for cp_ in cps:
            cp_.wait()
        prep_idx(0)
        prep_idx(1)
        pltpu.async_copy(tab_hbm.at[dsts[0]], t_bs[0], gsems[0])

        def six_body(i, carry):
            for b in range(6):
                chunk_step(6 * i + b, b % 2, b % 3)
            return carry
        lax.fori_loop(0, nchunk // 6, six_body, 0)
        for b in range(nchunk % 6):
            chunk_step(jnp.int32(6 * (nchunk // 6) + b), b % 2, b % 3)
        lastk = nchunk - 1
        pltpu.make_async_copy(t_bs[lastk % 2], acc.at[srcs[lastk % 3]],
                              asem).wait()

        plsc.subcore_barrier()

        @pl.when(s < NTD)
        def _drain_acc():
            sl = pl.ds(s * RPT, RPT)
            pltpu.sync_copy(acc.at[sl], acc_out.at[c, sl])

        if pair:
            @pl.when(s == 0)
            def _drain_rs():
                pltpu.sync_copy(rsacc, rs_out.at[c])

    return edge_pass


_edge_pass1 = _make_edge_pass(True)
_edge_pass2 = _make_edge_pass(False)


def kernel(x, adj, W0, W1, W2, W3, a0, a1, a2, a3, Wout, aout):
    f32 = jnp.float32
    wcat = jnp.concatenate([W0, W1, W2, W3], axis=1)
    asrc = jnp.zeros((256, 16), f32)
    adst = jnp.zeros((256, 16), f32)
    for h, a in enumerate([a0, a1, a2, a3]):
        asrc = asrc.at[h * NHID:(h + 1) * NHID, h].set(a[:NHID])
        adst = adst.at[h * NHID:(h + 1) * NHID, h].set(a[NHID:])
    wo128 = jnp.zeros((256, 64), f32).at[:, :NCLS].set(Wout)
    as2 = jnp.zeros((64, 16), f32).at[:NCLS, 0].set(aout[:NCLS])
    ad2 = jnp.zeros((64, 16), f32).at[:NCLS, 0].set(aout[NCLS:])

    hflat, fs16, fd16 = _pre(x, wcat, asrc, adst)

    def pack2(a, b):
        ab = lax.bitcast_convert_type(a.astype(jnp.bfloat16), jnp.uint16)
        bb = lax.bitcast_convert_type(b.astype(jnp.bfloat16), jnp.uint16)
        w = ab.astype(jnp.uint32) | (bb.astype(jnp.uint32) << 16)
        return lax.bitcast_convert_type(w, jnp.int32)

    logits1 = jnp.concatenate(
        [pack2(fs16[:, 0], fs16[:, 1]), pack2(fs16[:, 2], fs16[:, 3]),
         pack2(fd16[:, 0], fd16[:, 1]), pack2(fd16[:, 2], fd16[:, 3])])
    adjf = jnp.concatenate(
        [adj.reshape(2 * EE), jnp.zeros((CHUNK,), jnp.int32)])
    hp, rs1 = _edge_pass1(adjf, logits1, hflat.reshape(2 * NN, 128))
    rs4 = rs1.reshape(NC, 160 * 64, 2)[:, :NN, :].transpose(1, 0, 2).reshape(NN, 4)
    g128, gs16, gd16 = _mid(hp, rs4, wo128, as2, ad2)
    logits2 = jnp.concatenate([gs16[:, 0], gd16[:, 0]])
    acc2, _ = _edge_pass2(adjf, logits2, g128)
    return _post(acc2)


# speedup vs baseline: 1.4990x; 1.0355x over previous
import functools

import jax
import jax.numpy as jnp
from jax import lax
from jax.experimental import pallas as pl
from jax.experimental.pallas import tpu as pltpu
from jax.experimental.pallas import tpu_sc as plsc

NN = 10000
EE = 320000
NFEAT = 128
NHID = 64
NCLS = 40
ALPHA = 0.2
NC, NS, L = 2, 16, 16
CHUNK = 64
NG = CHUNK // L
RPT = 1000
NTD = NN // RPT
NB = 1000


def _elu(v):
    return jnp.where(v > 0, v, jnp.exp(jnp.minimum(v, 0.0)) - 1.0)


def _pre_body(x_ref, wc_ref, as_ref, ad_ref, hf_ref, fs_ref, fd_ref):
    h = jnp.dot(x_ref[...], wc_ref[...], preferred_element_type=jnp.float32)
    hf_ref[0] = h[:, :128]
    hf_ref[1] = h[:, 128:]
    fs_ref[...] = jnp.dot(h, as_ref[...], preferred_element_type=jnp.float32)
    fd_ref[...] = jnp.dot(h, ad_ref[...], preferred_element_type=jnp.float32)


def _pre(x, wcat, asrc, adst):
    return pl.pallas_call(
        _pre_body,
        grid=(NN // NB,),
        in_specs=[
            pl.BlockSpec((NB, NFEAT), lambda i: (i, 0)),
            pl.BlockSpec((NFEAT, 256), lambda i: (0, 0)),
            pl.BlockSpec((256, 16), lambda i: (0, 0)),
            pl.BlockSpec((256, 16), lambda i: (0, 0)),
        ],
        out_specs=[
            pl.BlockSpec((2, NB, 128), lambda i: (0, i, 0)),
            pl.BlockSpec((NB, 16), lambda i: (i, 0)),
            pl.BlockSpec((NB, 16), lambda i: (i, 0)),
        ],
        out_shape=[
            jax.ShapeDtypeStruct((2, NN, 128), jnp.float32),
            jax.ShapeDtypeStruct((NN, 16), jnp.float32),
            jax.ShapeDtypeStruct((NN, 16), jnp.float32),
        ],
    )(x, wcat, asrc, adst)


def _mid_body(hp_ref, rs_ref, wo_ref, as2_ref, ad2_ref, g_ref, gs_ref, gd_ref):
    parts = []
    for head in range(4):
        c, j = divmod(head, 2)
        hp = hp_ref[c][:, j * NHID:(j + 1) * NHID]
        denom = rs_ref[:, head][:, None] + 1e-9
        parts.append(_elu(hp / denom))
    x1 = jnp.concatenate(parts, axis=1)
    g = jnp.dot(x1, wo_ref[...], preferred_element_type=jnp.float32)
    col = lax.broadcasted_iota(jnp.int32, g.shape, 1)
    g_ref[...] = jnp.where(col == NCLS + 8, 1.0, g)
    gs_ref[...] = jnp.dot(g, as2_ref[...], preferred_element_type=jnp.float32)
    gd_ref[...] = jnp.dot(g, ad2_ref[...], preferred_element_type=jnp.float32)


def _mid(hp, rs, wo128, as2, ad2):
    return pl.pallas_call(
        _mid_body,
        grid=(NN // NB,),
        in_specs=[
            pl.BlockSpec((2, NB, 128), lambda i: (0, i, 0)),
            pl.BlockSpec((NB, 4), lambda i: (i, 0)),
            pl.BlockSpec((256, 64), lambda i: (0, 0)),
            pl.BlockSpec((64, 16), lambda i: (0, 0)),
            pl.BlockSpec((64, 16), lambda i: (0, 0)),
        ],
        out_specs=[
            pl.BlockSpec((NB, 64), lambda i: (i, 0)),
            pl.BlockSpec((NB, 16), lambda i: (i, 0)),
            pl.BlockSpec((NB, 16), lambda i: (i, 0)),
        ],
        out_shape=[
            jax.ShapeDtypeStruct((NN, 64), jnp.float32),
            jax.ShapeDtypeStruct((NN, 16), jnp.float32),
            jax.ShapeDtypeStruct((NN, 16), jnp.float32),
        ],
    )(hp, rs, wo128, as2, ad2)


def _post_body(acc_ref, o_ref):
    comb = acc_ref[0][:, :NCLS] + acc_ref[1][:, :NCLS]
    rsum = (acc_ref[0][:, NCLS + 8] + acc_ref[1][:, NCLS + 8])[:, None] + 1e-9
    o = _elu(comb / rsum)
    m = jnp.max(o, axis=1, keepdims=True)
    lse = jnp.log(jnp.sum(jnp.exp(o - m), axis=1, keepdims=True))
    o_ref[...] = o - m - lse


def _post(acc2):
    return pl.pallas_call(
        _post_body,
        grid=(NN // NB,),
        in_specs=[
            pl.BlockSpec((2, NB, 64), lambda i: (0, i, 0)),
        ],
        out_specs=pl.BlockSpec((NB, NCLS), lambda i: (i, 0)),
        out_shape=jax.ShapeDtypeStruct((NN, NCLS), jnp.float32),
    )(acc2)


def _make_edge_pass(pair):
    ept = EE // NS if pair else EE // (NC * NS)
    nchunk = -(-ept // CHUNK)
    nj = 2 if pair else 1
    tw = 128 if pair else 64
    nscale = 8 if pair else 4
    shift = 3
    nmask = 7
    lmul = 2
    nrs = 1280

    mesh = plsc.VectorSubcoreMesh(
        core_axis_name="c", subcore_axis_name="s",
        num_cores=NC, num_subcores=NS)

    @functools.partial(
        pl.kernel,
        out_type=[
            jax.ShapeDtypeStruct((NC, NN, tw), jnp.float32),
            jax.ShapeDtypeStruct((NC, nrs, L), jnp.float32),
        ],
        mesh=mesh,
        compiler_params=pltpu.CompilerParams(
            needs_layout_passes=False,
            use_tc_tiling_on_sc=False),
        scratch_types=[
            pltpu.VMEM((CHUNK,), jnp.int32),
            pltpu.VMEM((CHUNK,), jnp.int32),
            pltpu.VMEM((CHUNK,), jnp.int32),
            pltpu.VMEM((CHUNK,), jnp.int32),
            pltpu.VMEM((CHUNK,), jnp.int32),
            pltpu.VMEM((CHUNK,), jnp.int32),
            pltpu.VMEM((CHUNK,), jnp.int32),
            pltpu.VMEM((CHUNK,), jnp.int32),
            pltpu.VMEM((CHUNK,), jnp.int32),
            pltpu.VMEM((NN,), jnp.int32 if pair else jnp.float32),
            pltpu.VMEM((NN,), jnp.int32 if pair else jnp.float32),
            pltpu.VMEM((CHUNK, tw), jnp.float32),
            pltpu.VMEM((CHUNK, tw), jnp.float32),
            pltpu.VMEM((CHUNK, L), jnp.float32),
            pltpu.VMEM((CHUNK * 2 + L,), jnp.float32),
            pltpu.VMEM_SHARED((NN, tw), jnp.float32),
            pltpu.VMEM_SHARED((nrs, L), jnp.float32),
            pltpu.SemaphoreType.DMA,
            pltpu.SemaphoreType.DMA,
            pltpu.SemaphoreType.DMA,
            pltpu.SemaphoreType.DMA,
            pltpu.SemaphoreType.DMA,
        ],
    )
    def edge_pass(adj, logits_hbm, tab_hbm,
                  acc_out, rs_out,
                  src_v0, src_v1, src_v2, dst_v0, dst_v1, dst_v2,
                  srow_v0, srow_v1, srow_v2, fsT, fdT,
                  t_b0, t_b1, rs_b, e_bf, acc, rsacc,
                  gsem0, gsem1, asem, rsem, isem):
        srcs = (src_v0, src_v1, src_v2)
        dsts = (dst_v0, dst_v1, dst_v2)
        srows = (srow_v0, srow_v1, srow_v2)
        t_bs = (t_b0, t_b1)
        gsems = (gsem0, gsem1)
        c = lax.axis_index("c")
        s = lax.axis_index("s")

        if pair:
            pltpu.sync_copy(logits_hbm.at[pl.ds(c * NN, NN)], fsT)
            pltpu.sync_copy(logits_hbm.at[pl.ds((2 + c) * NN, NN)], fdT)
        else:
            pltpu.sync_copy(logits_hbm.at[pl.ds(0, NN)], fsT)
            pltpu.sync_copy(logits_hbm.at[pl.ds(NN, NN)], fdT)

        def z_body(i, cr):
            for j in range(tw // L):
                t_b0[i, pl.ds(j * L, L)] = jnp.zeros((L,), jnp.float32)
            if pair:
                rs_b[i] = jnp.zeros((L,), jnp.float32)
            return cr
        lax.fori_loop(0, CHUNK, z_body, 0)
        zsrc = t_b0

        def zacc_body(i, cr):
            m = s + 16 * i

            @pl.when(m < NN // CHUNK)
            def _():
                pltpu.sync_copy(zsrc, acc.at[pl.ds(m * CHUNK, CHUNK)])
            return cr
        lax.fori_loop(0, (NN // CHUNK + 15) // 16, zacc_body, 0)
        if NN % CHUNK:
            @pl.when(s == NS - 1)
            def _init_acc_tail():
                pltpu.sync_copy(zsrc.at[pl.ds(0, NN % CHUNK)],
                                acc.at[pl.ds(NN - NN % CHUNK, NN % CHUNK)])

        if pair:
            def zrs_body(i, cr):
                m = s + 16 * i

                @pl.when(m < nrs // CHUNK)
                def _():
                    pltpu.sync_copy(rs_b, rsacc.at[pl.ds(m * CHUNK, CHUNK)])
                return cr
            lax.fori_loop(0, (nrs // CHUNK + 15) // 16, zrs_body, 0)
        plsc.subcore_barrier()

        lane = lax.iota(jnp.int32, L)
        tile_e0 = s * ept if pair else c * (EE // NC) + s * ept
        zeros16 = jnp.zeros((L,), jnp.float32)
        zidx = jnp.zeros((L,), jnp.int32)
        bdnums = lax.GatherDimensionNumbers(
            offset_dims=(), collapsed_slice_dims=(0,), start_index_map=(0,))

        def bcast(v, iv):
            return lax.gather(v, iv[:, None], bdnums, (1,),
                              mode=lax.GatherScatterMode.PROMISE_IN_BOUNDS)

        def issue_idx(k, st):
            base = tile_e0 + k * CHUNK
            return (pltpu.async_copy(adj.at[pl.ds(base, CHUNK)],
                                     srcs[st], isem),
                    pltpu.async_copy(adj.at[pl.ds(EE + base, CHUNK)],
                                     dsts[st], isem))

        def prep_idx(st):
            for j in range(NG):
                sl = pl.ds(j * L, L)
                srows[st][sl] = lax.shift_right_logical(srcs[st][sl], shift)
            if pair:
                for j in range(NG):
                    sl = pl.ds(j * L, L)
                    dsts[st][sl] = dsts[st][sl] + c * NN

        def chunk_step(k, c2, c3):
            p2, p3, n3 = 1 - c2, (c3 + 2) % 3, (c3 + 1) % 3
            t_b = t_bs[c2]

            def e_body(g, cr):
                sidx = srcs[c3][pl.ds(g * L, L)]
                didx = dsts[c3][pl.ds(g * L, L)]
                if pair:
                    didx = didx - c * NN
                rowi = g * L + lane
                cbase = (sidx & nmask) * lmul
                ebase = rowi * 2
                ws = plsc.load_gather(fsT, [sidx])
                wd = plsc.load_gather(fdT, [didx])
                if pair:
                    hi = jnp.int32(-65536)
                    zs = (plsc.bitcast(jnp.left_shift(ws, 16), jnp.float32),
                          plsc.bitcast(jnp.bitwise_and(ws, hi), jnp.float32))
                    zd = (plsc.bitcast(jnp.left_shift(wd, 16), jnp.float32),
                          plsc.bitcast(jnp.bitwise_and(wd, hi), jnp.float32))
                else:
                    zs = (ws,)
                    zd = (wd,)
                inb = (k * CHUNK + rowi) < ept
                for j in range(nj):
                    z = zs[j] + zd[j]
                    ev = jnp.where(inb, jnp.exp(-jnp.maximum(z, z * ALPHA)),
                                   0.0)
                    if pair:
                        plsc.store_scatter(rs_b, [rowi, cbase + j], ev)
                    plsc.store_scatter(e_bf, [ebase + j], ev)
                return cr
            for g_ in range(NG):
                e_body(g_, 0)

            if pair:
                rcp = pltpu.async_copy(rs_b, rsacc.at[srows[c3]], rsem,
                                       add=True)

            @pl.when(k > 0)
            def _():
                pltpu.make_async_copy(t_bs[p2], acc.at[srcs[p3]],
                                      asem).wait()

            @pl.when(k + 1 < nchunk)
            def _():
                pltpu.async_copy(tab_hbm.at[dsts[n3]], t_bs[p2],
                                 gsems[p2])

            @pl.when(k + 2 < nchunk)
            def _():
                issue_idx(k + 2, p3)

            pltpu.make_async_copy(tab_hbm.at[dsts[c3]], t_b,
                                  gsems[c2]).wait()

            def s_body(i, cr):
                ev = e_bf[pl.ds(i * 2, L)]
                eA = bcast(ev, zidx)
                eB = bcast(ev, zidx + 1) if pair else eA
                for j in range(nscale):
                    sl = pl.ds(j * L, L)
                    ee = eA if (not pair or j < 4) else eB
                    t_b[i, sl] = t_b[i, sl] * ee
                return cr
            lax.fori_loop(0, CHUNK, s_body, 0, unroll=8)

            pltpu.async_copy(t_b, acc.at[srcs[c3]], asem, add=True)

            if pair:
                rcp.wait()

                def uz_body(g, cr):
                    sidx = srcs[c3][pl.ds(g * L, L)]
                    rowi = g * L + lane
                    cbase = (sidx & nmask) * lmul
                    for j in range(nj):
                        plsc.store_scatter(rs_b, [rowi, cbase + j], zeros16)
                    return cr
                for g_ in range(NG):
                    uz_body(g_, 0)

            @pl.when(k + 2 < nchunk)
            def _():
                base = tile_e0 + (k + 2) * CHUNK
                pltpu.make_async_copy(adj.at[pl.ds(base, CHUNK)],
                                      srcs[p3], isem).wait()
                pltpu.make_async_copy(adj.at[pl.ds(EE + base, CHUNK)],
                                      dsts[p3], isem).wait()
                prep_idx(p3)

        cps = issue_idx(jnp.int32(0), 0) + issue_idx(jnp.int32(1), 1)
        for cp_ in cps:
            cp_.wait()
        prep_idx(0)
        prep_idx(1)
        pltpu.async_copy(tab_hbm.at[dsts[0]], t_bs[0], gsems[0])

        def six_body(i, carry):
            for b in range(6):
                chunk_step(6 * i + b, b % 2, b % 3)
            return carry
        lax.fori_loop(0, nchunk // 6, six_body, 0)
        for b in range(nchunk % 6):
            chunk_step(jnp.int32(6 * (nchunk // 6) + b), b % 2, b % 3)
        lastk = nchunk - 1
        pltpu.make_async_copy(t_bs[lastk % 2], acc.at[srcs[lastk % 3]],
                              asem).wait()

        plsc.subcore_barrier()

        @pl.when(s < NTD)
        def _drain_acc():
            sl = pl.ds(s * RPT, RPT)
            pltpu.sync_copy(acc.at[sl], acc_out.at[c, sl])

        if pair:
            @pl.when(s == 0)
            def _drain_rs():
                pltpu.sync_copy(rsacc, rs_out.at[c])

    return edge_pass


_edge_pass1 = _make_edge_pass(True)
_edge_pass2 = _make_edge_pass(False)


def kernel(x, adj, W0, W1, W2, W3, a0, a1, a2, a3, Wout, aout):
    f32 = jnp.float32
    wcat = jnp.concatenate([W0, W1, W2, W3], axis=1)
    asrc = jnp.zeros((256, 16), f32)
    adst = jnp.zeros((256, 16), f32)
    for h, a in enumerate([a0, a1, a2, a3]):
        asrc = asrc.at[h * NHID:(h + 1) * NHID, h].set(a[:NHID])
        adst = adst.at[h * NHID:(h + 1) * NHID, h].set(a[NHID:])
    wo128 = jnp.zeros((256, 64), f32).at[:, :NCLS].set(Wout)
    as2 = jnp.zeros((64, 16), f32).at[:NCLS, 0].set(aout[:NCLS])
    ad2 = jnp.zeros((64, 16), f32).at[:NCLS, 0].set(aout[NCLS:])

    hflat, fs16, fd16 = _pre(x, wcat, asrc, adst)

    def pack2(a, b):
        ab = lax.bitcast_convert_type(a.astype(jnp.bfloat16), jnp.uint16)
        bb = lax.bitcast_convert_type(b.astype(jnp.bfloat16), jnp.uint16)
        w = ab.astype(jnp.uint32) | (bb.astype(jnp.uint32) << 16)
        return lax.bitcast_convert_type(w, jnp.int32)

    logits1 = jnp.concatenate(
        [pack2(fs16[:, 0], fs16[:, 1]), pack2(fs16[:, 2], fs16[:, 3]),
         pack2(fd16[:, 0], fd16[:, 1]), pack2(fd16[:, 2], fd16[:, 3])])
    adjf = jnp.concatenate(
        [adj.reshape(2 * EE), jnp.zeros((CHUNK,), jnp.int32)])
    hp, rs1 = _edge_pass1(adjf, logits1, hflat.reshape(2 * NN, 128))
    rs4 = rs1.reshape(NC, 160 * 64, 2)[:, :NN, :].transpose(1, 0, 2).reshape(NN, 4)
    g128, gs16, gd16 = _mid(hp, rs4, wo128, as2, ad2)
    logits2 = jnp.concatenate([gs16[:, 0], gd16[:, 0]])
    acc2, _ = _edge_pass2(adjf, logits2, g128)
    return _post(acc2)
